# baseline (speedup 1.0000x reference)
import functools

import jax
import jax.numpy as jnp
import numpy as np
from jax.experimental import pallas as pl
from jax.experimental.pallas import tpu as pltpu
from jax.experimental.pallas import tpu_sc as plsc

_B = 2
_KP = 16
_DM = 128
_SQRT_DM = np.float32(np.sqrt(128.0))

_SC_NC = 2
_SC_NS = 16
_SC_NW = _SC_NC * _SC_NS


def _sc_gather(table, idx, chunk):
    bn = idx.shape[0]
    d = table.shape[1]
    per_w = bn // (chunk * _SC_NW)
    mesh = plsc.VectorSubcoreMesh(core_axis_name="c", subcore_axis_name="s",
                                  num_cores=_SC_NC, num_subcores=_SC_NS)

    def body(table_hbm, idx_hbm, out_hbm, idx_v, rows_v, sem):
        wid = jax.lax.axis_index("s") * _SC_NC + jax.lax.axis_index("c")
        for j in range(per_w):
            base = (wid * per_w + j) * chunk
            pltpu.sync_copy(idx_hbm.at[pl.ds(base, chunk)], idx_v)
            pltpu.async_copy(table_hbm.at[idx_v], rows_v, sem).wait()
            pltpu.sync_copy(rows_v, out_hbm.at[pl.ds(base, chunk)])

    f = pl.kernel(
        body,
        out_type=jax.ShapeDtypeStruct((bn, d), jnp.float32),
        mesh=mesh,
        scratch_types=[pltpu.VMEM((chunk,), jnp.int32),
                       pltpu.VMEM((chunk, d), jnp.float32),
                       pltpu.SemaphoreType.DMA],
    )
    return f(table, idx)


def _sc_chunk(bn):
    for c in (128, 64, 32, 16, 8):
        if bn % (c * _SC_NW) == 0:
            return c
    return 0


def _rep_spec(shape):
    nd = len(shape)
    return pl.BlockSpec(shape, lambda *_: (0,) * nd)


def _batch_spec(shape):
    nd = len(shape)
    return pl.BlockSpec((None,) + shape, lambda b: (b,) + (0,) * nd)


def _fps_body(npoint, r, c, xyzR_ref, xyzD_ref, out_ref):
    n = r * c
    x = xyzD_ref[0:r, :]
    y = xyzD_ref[r:2 * r, :]
    z = xyzD_ref[2 * r:3 * r, :]
    idx2d = (jax.lax.broadcasted_iota(jnp.int32, (r, c), 0) * c
             + jax.lax.broadcasted_iota(jnp.int32, (r, c), 1))

    def body(i, carry):
        dist_min, far = carry
        out_ref[pl.ds(i, 1), :] = jnp.reshape(far, (1, 1))
        row = xyzR_ref[pl.ds(far, 1), :]
        dx = x - row[:, 0:1]
        dy = y - row[:, 1:2]
        dz = z - row[:, 2:3]
        dist = dx * dx + dy * dy + dz * dz
        dist_min = jnp.minimum(dist_min, dist)
        m = jnp.max(dist_min)
        sel = jnp.where(dist_min == m, idx2d, n)
        far2 = jnp.min(sel)
        return dist_min, far2

    init = (jnp.full((r, c), 1e10, dtype=jnp.float32), jnp.int32(0))
    jax.lax.fori_loop(0, npoint, body, init)


def _fps(xyz, npoint):
    b, n, _ = xyz.shape
    r = max(n // 128, 1)
    c = min(n, 128)
    xyz_d = jnp.transpose(xyz, (0, 2, 1)).reshape(b, 3 * r, c)
    out = pl.pallas_call(
        functools.partial(_fps_body, npoint, r, c),
        grid=(b,),
        in_specs=[_batch_spec((n, 3)), _batch_spec((3 * r, c))],
        out_specs=_batch_spec((npoint, 1)),
        out_shape=jax.ShapeDtypeStruct((b, npoint, 1), jnp.int32),
    )(xyz, xyz_d)
    return out


def _knn_body(k, has_qidx, global_ofs, *refs):
    if has_qidx:
        xyzR_ref, xyzT_ref, qidx_ref, out_ref = refs
    else:
        xyzR_ref, xyzT_ref, out_ref = refs
    n = xyzT_ref.shape[-1]
    if has_qidx:
        nq = qidx_ref.shape[0]
        lane_q = jax.lax.broadcasted_iota(jnp.int32, (nq, n), 1)
        oh = (lane_q == qidx_ref[:, :]).astype(jnp.float32)
        q = jnp.dot(oh, xyzR_ref[...], preferred_element_type=jnp.float32)
    else:
        nq = xyzR_ref.shape[0]
        q = xyzR_ref[...]
    qx = q[:, 0:1]
    qy = q[:, 1:2]
    qz = q[:, 2:3]
    dx = qx - xyzT_ref[0:1, :]
    dy = qy - xyzT_ref[1:2, :]
    dz = qz - xyzT_ref[2:3, :]
    d = dx * dx + dy * dy + dz * dz
    lane = jax.lax.broadcasted_iota(jnp.int32, (nq, n), 1)
    big = jnp.float32(np.inf)
    ofs = pl.program_id(0) * n if global_ofs else 0
    for j in range(k):
        m = jnp.min(d, axis=1, keepdims=True)
        sel = jnp.where(d == m, lane, n)
        amin = jnp.min(sel, axis=1, keepdims=True)
        out_ref[:, pl.ds(j, 1)] = amin + ofs
        d = jnp.where(lane == amin, big, d)


def _knn_self(xyz, k, global_ofs=False):
    b, n, _ = xyz.shape
    xyz_t = jnp.transpose(xyz, (0, 2, 1))
    return pl.pallas_call(
        functools.partial(_knn_body, k, False, global_ofs),
        grid=(b,),
        in_specs=[_batch_spec((n, 3)), _batch_spec((3, n))],
        out_specs=_batch_spec((n, k)),
        out_shape=jax.ShapeDtypeStruct((b, n, k), jnp.int32),
    )(xyz, xyz_t)


def _knn_fps(xyz, qidx, k):
    b, n, _ = xyz.shape
    nq = qidx.shape[1]
    xyz_t = jnp.transpose(xyz, (0, 2, 1))
    return pl.pallas_call(
        functools.partial(_knn_body, k, True, False),
        grid=(b,),
        in_specs=[_batch_spec((n, 3)), _batch_spec((3, n)),
                  _batch_spec((nq, 1))],
        out_specs=_batch_spec((nq, k)),
        out_shape=jax.ShapeDtypeStruct((b, nq, k), jnp.int32),
    )(xyz, xyz_t, qidx)


def _tb_pre_body(xyzR_ref, f_ref, fc1w_ref, fc1b_ref, wq_ref, wk_ref,
                 wv_ref, d1w_ref, q_ref, t_ref):
    x = jnp.dot(f_ref[...], fc1w_ref[...],
                preferred_element_type=jnp.float32) + fc1b_ref[...]
    q_ref[...] = jnp.dot(x, wq_ref[...], preferred_element_type=jnp.float32)
    t_ref[:, 0:128] = jnp.dot(x, wk_ref[...],
                              preferred_element_type=jnp.float32)
    t_ref[:, 128:256] = jnp.dot(x, wv_ref[...],
                                preferred_element_type=jnp.float32)
    t_ref[:, 256:384] = jnp.dot(xyzR_ref[...], d1w_ref[...],
                                preferred_element_type=jnp.float32)


def _tb_post_body(k, tile, d1b_ref, d2w_ref, d2b_ref, g1w_ref, g1b_ref,
                  g2w_ref, g2b_ref, fc2w_ref, fc2b_ref, t_ref, q_ref,
                  knn_ref, pre_ref, out_ref, a_sc, w_sc):
    n = t_ref.shape[0]
    tid = pl.program_id(1)
    pq = t_ref[pl.ds(tid * tile, tile), 256:384]
    qv = q_ref[...]
    knn = knn_ref[...]
    lane = jax.lax.broadcasted_iota(jnp.int32, (tile, n), 1)
    table = t_ref[...]
    t_hi = table.astype(jnp.bfloat16)
    r1 = table - t_hi.astype(jnp.float32)
    t_mid = r1.astype(jnp.bfloat16)
    t_lo = (r1 - t_mid.astype(jnp.float32)).astype(jnp.bfloat16)
    for j in range(k):
        idx = knn[:, j:j + 1]
        oh = (lane == idx).astype(jnp.bfloat16)
        g = (jnp.dot(oh, t_hi, preferred_element_type=jnp.float32)
             + jnp.dot(oh, t_mid, preferred_element_type=jnp.float32)
             + jnp.dot(oh, t_lo, preferred_element_type=jnp.float32))
        xk = g[:, 0:128]
        xv = g[:, 128:256]
        pg = g[:, 256:384]
        pos = jnp.maximum(pq - pg + d1b_ref[...], 0.0)
        pos = jnp.dot(pos, d2w_ref[...],
                      preferred_element_type=jnp.float32) + d2b_ref[...]
        u = qv - xk + pos
        a = jnp.maximum(jnp.dot(u, g1w_ref[...],
                                preferred_element_type=jnp.float32)
                        + g1b_ref[...], 0.0)
        a = jnp.dot(a, g2w_ref[...],
                    preferred_element_type=jnp.float32) + g2b_ref[...]
        a_sc[j] = a / _SQRT_DM
        w_sc[j] = xv + pos
    m = a_sc[0]
    for j in range(1, k):
        m = jnp.maximum(m, a_sc[j])
    s = jnp.zeros((tile, _DM), jnp.float32)
    acc = jnp.zeros((tile, _DM), jnp.float32)
    for j in range(k):
        e = jnp.exp(a_sc[j] - m)
        s = s + e
        acc = acc + e * w_sc[j]
    res = acc / s
    out_ref[...] = (jnp.dot(res, fc2w_ref[...],
                            preferred_element_type=jnp.float32)
                    + fc2b_ref[...] + pre_ref[...])


def _tb_post_g_body(k, tile, d1b_ref, d2w_ref, d2b_ref, g1w_ref, g1b_ref,
                    g2w_ref, g2b_ref, fc2w_ref, fc2b_ref, g_ref, tq_ref,
                    q_ref, pre_ref, out_ref, a_sc, w_sc):
    pq = tq_ref[:, 256:384]
    qv = q_ref[...]
    for j in range(k):
        base = j * 384
        xk = g_ref[:, base:base + 128]
        xv = g_ref[:, base + 128:base + 256]
        pg = g_ref[:, base + 256:base + 384]
        pos = jnp.maximum(pq - pg + d1b_ref[...], 0.0)
        pos = jnp.dot(pos, d2w_ref[...],
                      preferred_element_type=jnp.float32) + d2b_ref[...]
        u = qv - xk + pos
        a = jnp.maximum(jnp.dot(u, g1w_ref[...],
                                preferred_element_type=jnp.float32)
                        + g1b_ref[...], 0.0)
        a = jnp.dot(a, g2w_ref[...],
                    preferred_element_type=jnp.float32) + g2b_ref[...]
        a_sc[j] = a / _SQRT_DM
        w_sc[j] = xv + pos
    m = a_sc[0]
    for j in range(1, k):
        m = jnp.maximum(m, a_sc[j])
    s = jnp.zeros((tile, _DM), jnp.float32)
    acc = jnp.zeros((tile, _DM), jnp.float32)
    for j in range(k):
        e = jnp.exp(a_sc[j] - m)
        s = s + e
        acc = acc + e * w_sc[j]
    res = acc / s
    out_ref[...] = (jnp.dot(res, fc2w_ref[...],
                            preferred_element_type=jnp.float32)
                    + fc2b_ref[...] + pre_ref[...])


def _tb(p, xyz, feats, knn, use_sc=False):
    b, n, d_in = feats.shape
    k = knn.shape[2]
    fc1w, fc1b = p['fc1']
    d1w, d1b = p['d1']
    d2w, d2b = p['d2']
    g1w, g1b = p['g1']
    g2w, g2b = p['g2']
    fc2w, fc2b = p['fc2']
    q, t = pl.pallas_call(
        _tb_pre_body,
        grid=(b,),
        in_specs=[_batch_spec((n, 3)), _batch_spec((n, d_in)),
                  _rep_spec(fc1w.shape), _rep_spec((1, _DM)),
                  _rep_spec(p['wq'].shape), _rep_spec(p['wk'].shape),
                  _rep_spec(p['wv'].shape), _rep_spec(d1w.shape)],
        out_specs=[_batch_spec((n, _DM)), _batch_spec((n, 384))],
        out_shape=[jax.ShapeDtypeStruct((b, n, _DM), jnp.float32),
                   jax.ShapeDtypeStruct((b, n, 384), jnp.float32)],
    )(xyz, feats, fc1w, fc1b.reshape(1, -1), p['wq'], p['wk'], p['wv'], d1w)

    if use_sc:
        chunk = _sc_chunk(b * n * k)
        g = _sc_gather(t.reshape(b * n, 384), knn.reshape(-1), chunk)
        gr = g.reshape(b, n, k * 384)
        tile = min(n, 128)
        nt = n // tile
        out = pl.pallas_call(
            functools.partial(_tb_post_g_body, k, tile),
            grid=(b, nt),
            in_specs=[_rep_spec((1, _DM)), _rep_spec(d2w.shape),
                      _rep_spec((1, _DM)), _rep_spec(g1w.shape),
                      _rep_spec((1, _DM)), _rep_spec(g2w.shape),
                      _rep_spec((1, _DM)), _rep_spec(fc2w.shape),
                      _rep_spec((1, d_in)),
                      pl.BlockSpec((None, tile, k * 384),
                                   lambda b_, t_: (b_, t_, 0)),
                      pl.BlockSpec((None, tile, 384),
                                   lambda b_, t_: (b_, t_, 0)),
                      pl.BlockSpec((None, tile, _DM),
                                   lambda b_, t_: (b_, t_, 0)),
                      pl.BlockSpec((None, tile, d_in),
                                   lambda b_, t_: (b_, t_, 0))],
            out_specs=pl.BlockSpec((None, tile, d_in),
                                   lambda b_, t_: (b_, t_, 0)),
            out_shape=jax.ShapeDtypeStruct((b, n, d_in), jnp.float32),
            scratch_shapes=[pltpu.VMEM((k, tile, _DM), jnp.float32),
                            pltpu.VMEM((k, tile, _DM), jnp.float32)],
        )(d1b.reshape(1, -1), d2w, d2b.reshape(1, -1), g1w,
          g1b.reshape(1, -1), g2w, g2b.reshape(1, -1), fc2w,
          fc2b.reshape(1, -1), gr, t, q, feats)
        return out

    tile = min(n, 256)
    nt = n // tile
    out = pl.pallas_call(
        functools.partial(_tb_post_body, k, tile),
        grid=(b, nt),
        in_specs=[_rep_spec((1, _DM)), _rep_spec(d2w.shape),
                  _rep_spec((1, _DM)), _rep_spec(g1w.shape),
                  _rep_spec((1, _DM)), _rep_spec(g2w.shape),
                  _rep_spec((1, _DM)), _rep_spec(fc2w.shape),
                  _rep_spec((1, d_in)),
                  pl.BlockSpec((None, n, 384), lambda b_, t_: (b_, 0, 0)),
                  pl.BlockSpec((None, tile, _DM), lambda b_, t_: (b_, t_, 0)),
                  pl.BlockSpec((None, tile, k), lambda b_, t_: (b_, t_, 0)),
                  pl.BlockSpec((None, tile, d_in), lambda b_, t_: (b_, t_, 0))],
        out_specs=pl.BlockSpec((None, tile, d_in), lambda b_, t_: (b_, t_, 0)),
        out_shape=jax.ShapeDtypeStruct((b, n, d_in), jnp.float32),
        scratch_shapes=[pltpu.VMEM((k, tile, _DM), jnp.float32),
                        pltpu.VMEM((k, tile, _DM), jnp.float32)],
    )(d1b.reshape(1, -1), d2w, d2b.reshape(1, -1), g1w, g1b.reshape(1, -1),
      g2w, g2b.reshape(1, -1), fc2w, fc2b.reshape(1, -1), t, q, knn, feats)
    return out


def _td_body(k, xyzR_ref, f_ref, fps_ref, knn_ref, l1wx_ref, l1wf_ref,
             l1b_ref, l2w_ref, l2b_ref, nxyz_ref, out_ref):
    n = xyzR_ref.shape[0]
    npt = fps_ref.shape[0]
    c_out = l2w_ref.shape[0]
    lane = jax.lax.broadcasted_iota(jnp.int32, (npt, n), 1)
    oh_fps = (lane == fps_ref[:, :]).astype(jnp.float32)
    new_xyz = jnp.dot(oh_fps, xyzR_ref[...],
                      preferred_element_type=jnp.float32)
    nxyz_ref[...] = new_xyz
    knn = knn_ref[...]
    m = jnp.full((npt, c_out), -jnp.inf, jnp.float32)
    for j in range(k):
        idx = knn[:, j:j + 1]
        oh = (lane == idx).astype(jnp.float32)
        gx = jnp.dot(oh, xyzR_ref[...],
                     preferred_element_type=jnp.float32) - new_xyz
        gf = jnp.dot(oh, f_ref[...], preferred_element_type=jnp.float32)
        h = (jnp.dot(gx, l1wx_ref[...], preferred_element_type=jnp.float32)
             + jnp.dot(gf, l1wf_ref[...], preferred_element_type=jnp.float32)
             + l1b_ref[...])
        h = jnp.maximum(h, 0.0)
        h = jnp.dot(h, l2w_ref[...],
                    preferred_element_type=jnp.float32) + l2b_ref[...]
        h = jnp.maximum(h, 0.0)
        m = jnp.maximum(m, h)
    out_ref[...] = m


def _td(p, xyz, feats, fps, knn):
    b, n, c_in = feats.shape
    npt = fps.shape[1]
    k = knn.shape[2]
    l1w, l1b = p['l1']
    l2w, l2b = p['l2']
    c_out = l2w.shape[1]
    nxyz, f_out = pl.pallas_call(
        functools.partial(_td_body, k),
        grid=(b,),
        in_specs=[_batch_spec((n, 3)), _batch_spec((n, c_in)),
                  _batch_spec((npt, 1)), _batch_spec((npt, k)),
                  _rep_spec((3, c_out)), _rep_spec((c_in, c_out)),
                  _rep_spec((1, c_out)), _rep_spec(l2w.shape),
                  _rep_spec((1, c_out))],
        out_specs=[_batch_spec((npt, 3)), _batch_spec((npt, c_out))],
        out_shape=[jax.ShapeDtypeStruct((b, npt, 3), jnp.float32),
                   jax.ShapeDtypeStruct((b, npt, c_out), jnp.float32)],
    )(xyz, feats, fps, knn, l1w[:3], l1w[3:], l1b.reshape(1, -1),
      l2w, l2b.reshape(1, -1))
    return nxyz, f_out


def _tu_body(fc_ref, xycR_ref, xycT_ref, ff_ref, xyf_ref, w1_ref, b1_ref,
             w2_ref, b2_ref, out_ref):
    nc = xycR_ref.shape[0]
    nf = xyf_ref.shape[0]
    f1 = jnp.maximum(jnp.dot(fc_ref[...], w1_ref[...],
                             preferred_element_type=jnp.float32)
                     + b1_ref[...], 0.0)
    f2 = jnp.maximum(jnp.dot(ff_ref[...], w2_ref[...],
                             preferred_element_type=jnp.float32)
                     + b2_ref[...], 0.0)
    dx = xyf_ref[:, 0:1] - xycT_ref[0:1, :]
    dy = xyf_ref[:, 1:2] - xycT_ref[1:2, :]
    dz = xyf_ref[:, 2:3] - xycT_ref[2:3, :]
    d = dx * dx + dy * dy + dz * dz
    lane = jax.lax.broadcasted_iota(jnp.int32, (nf, nc), 1)
    big = jnp.float32(np.inf)
    ws = []
    idxs = []
    for j in range(3):
        m = jnp.min(d, axis=1, keepdims=True)
        sel = jnp.where(d == m, lane, nc)
        amin = jnp.min(sel, axis=1, keepdims=True)
        ws.append(1.0 / jnp.maximum(m, 1e-10))
        idxs.append(amin)
        d = jnp.where(lane == amin, big, d)
    wsum = (ws[0] + ws[1]) + ws[2]
    acc = None
    for j in range(3):
        oh = (lane == idxs[j]).astype(jnp.float32)
        fj = jnp.dot(oh, f1, preferred_element_type=jnp.float32)
        term = (ws[j] / wsum) * fj
        acc = term if acc is None else acc + term
    out_ref[...] = acc + f2


def _tu(p, f_coarse, xyz_coarse, f_fine, xyz_fine):
    b, nc, _ = xyz_coarse.shape
    nf = xyz_fine.shape[1]
    w1, b1 = p['fc1']
    w2, b2 = p['fc2']
    d = w1.shape[1]
    xyc_t = jnp.transpose(xyz_coarse, (0, 2, 1))
    return pl.pallas_call(
        _tu_body,
        grid=(b,),
        in_specs=[_batch_spec(f_coarse.shape[1:]), _batch_spec((nc, 3)),
                  _batch_spec((3, nc)), _batch_spec(f_fine.shape[1:]),
                  _batch_spec((nf, 3)), _rep_spec(w1.shape),
                  _rep_spec((1, d)), _rep_spec(w2.shape), _rep_spec((1, d))],
        out_specs=_batch_spec((nf, d)),
        out_shape=jax.ShapeDtypeStruct((b, nf, d), jnp.float32),
    )(f_coarse, xyz_coarse, xyc_t, f_fine, xyz_fine, w1,
      b1.reshape(1, -1), w2, b2.reshape(1, -1))


def _mlp_body(relus, nlayer, *refs):
    x_ref = refs[0]
    out_ref = refs[-1]
    h = x_ref[...]
    for i in range(nlayer):
        w_ref = refs[1 + 2 * i]
        b_ref = refs[2 + 2 * i]
        h = jnp.dot(h, w_ref[...],
                    preferred_element_type=jnp.float32) + b_ref[...]
        if relus[i]:
            h = jnp.maximum(h, 0.0)
    out_ref[...] = h


def _mlp(x, layers, relus):
    b, n, _ = x.shape
    nlayer = len(layers)
    args = [x]
    specs = [_batch_spec(x.shape[1:])]
    for (w, bias) in layers:
        args.append(w)
        args.append(bias.reshape(1, -1))
        specs.append(_rep_spec(w.shape))
        specs.append(_rep_spec((1, w.shape[1])))
    d_out = layers[-1][0].shape[1]
    return pl.pallas_call(
        functools.partial(_mlp_body, relus, nlayer),
        grid=(b,),
        in_specs=specs,
        out_specs=_batch_spec((n, d_out)),
        out_shape=jax.ShapeDtypeStruct((b, n, d_out), jnp.float32),
    )(*args)


def _tb_stage(p, xyz, feats):
    n = xyz.shape[1]
    k = min(_KP, n)
    use_sc = False
    knn = _knn_self(xyz, k, global_ofs=use_sc)
    return _tb(p, xyz, feats, knn, use_sc=use_sc)


def kernel(x, params):
    xb = jnp.transpose(x, (0, 2, 1))
    xyz = xb
    f = _mlp(xb, [params['bb_fc1a'], params['bb_fc1b']], [True, False])
    f = _tb_stage(params['bb_tb0'], xyz, f)
    fac = [(f, xyz)]
    npts = xyz.shape[1]
    for i in range(4):
        npts //= 4
        fps = _fps(xyz, npts)
        knn_d = _knn_fps(xyz, fps, _KP)
        xyz, f = _td(params['bb_td'][i], xyz, f, fps, knn_d)
        f = _tb_stage(params['bb_tbs'][i], xyz, f)
        fac.append((f, xyz))
    feature, coord = fac[-1]
    h = _mlp(feature, [params['mlp2a'], params['mlp2b'], params['mlp2c']],
             [True, True, False])
    feature = _tb_stage(params['t2'], coord, h)
    for i in range(4):
        f_fine, c_fine = fac[-i - 2]
        feature = _tu(params['tu'][i], feature, coord, f_fine, c_fine)
        coord = c_fine
        feature = _tb_stage(params['tbu'][i], coord, feature)
    h = _mlp(feature, [params['mlp3a'], params['mlp3b'], params['mlp3c']],
             [True, True, False])
    return h


# speedup vs baseline: 1.2119x; 1.2119x over previous
import functools

import jax
import jax.numpy as jnp
import numpy as np
from jax.experimental import pallas as pl
from jax.experimental.pallas import tpu as pltpu
from jax.experimental.pallas import tpu_sc as plsc

_B = 2
_KP = 16
_DM = 128
_SQRT_DM = np.float32(np.sqrt(128.0))

_SC_NC = 2
_SC_NS = 16
_SC_NW = _SC_NC * _SC_NS


def _sc_gather(table, idx, chunk):
    bn = idx.shape[0]
    d = table.shape[1]
    per_w = bn // (chunk * _SC_NW)
    mesh = plsc.VectorSubcoreMesh(core_axis_name="c", subcore_axis_name="s",
                                  num_cores=_SC_NC, num_subcores=_SC_NS)

    def body(table_hbm, idx_hbm, out_hbm, idx_v, rows_v, sem):
        wid = jax.lax.axis_index("s") * _SC_NC + jax.lax.axis_index("c")
        for j in range(per_w):
            base = (wid * per_w + j) * chunk
            pltpu.sync_copy(idx_hbm.at[pl.ds(base, chunk)], idx_v)
            pltpu.async_copy(table_hbm.at[idx_v], rows_v, sem).wait()
            pltpu.sync_copy(rows_v, out_hbm.at[pl.ds(base, chunk)])

    f = pl.kernel(
        body,
        out_type=jax.ShapeDtypeStruct((bn, d), jnp.float32),
        mesh=mesh,
        scratch_types=[pltpu.VMEM((chunk,), jnp.int32),
                       pltpu.VMEM((chunk, d), jnp.float32),
                       pltpu.SemaphoreType.DMA],
    )
    return f(table, idx)


def _sc_chunk(bn):
    for c in (128, 64, 32, 16, 8):
        if bn % (c * _SC_NW) == 0:
            return c
    return 0


def _rep_spec(shape):
    nd = len(shape)
    return pl.BlockSpec(shape, lambda *_: (0,) * nd)


def _batch_spec(shape):
    nd = len(shape)
    return pl.BlockSpec((None,) + shape, lambda b: (b,) + (0,) * nd)


def _fps_body(npoint, r, c, xyzR_ref, xyzD_ref, out_ref):
    n = r * c
    x = xyzD_ref[0:r, :]
    y = xyzD_ref[r:2 * r, :]
    z = xyzD_ref[2 * r:3 * r, :]
    idx2d = (jax.lax.broadcasted_iota(jnp.int32, (r, c), 0) * c
             + jax.lax.broadcasted_iota(jnp.int32, (r, c), 1))

    def body(i, carry):
        dist_min, far = carry
        out_ref[pl.ds(i, 1), :] = jnp.reshape(far, (1, 1))
        row = xyzR_ref[pl.ds(far, 1), :]
        dx = x - row[:, 0:1]
        dy = y - row[:, 1:2]
        dz = z - row[:, 2:3]
        dist = dx * dx + dy * dy + dz * dz
        dist_min = jnp.minimum(dist_min, dist)
        m = jnp.max(dist_min)
        sel = jnp.where(dist_min == m, idx2d, n)
        far2 = jnp.min(sel)
        return dist_min, far2

    init = (jnp.full((r, c), 1e10, dtype=jnp.float32), jnp.int32(0))
    jax.lax.fori_loop(0, npoint, body, init)


def _fps(xyz, npoint):
    b, n, _ = xyz.shape
    r = max(n // 128, 1)
    c = min(n, 128)
    xyz_d = jnp.transpose(xyz, (0, 2, 1)).reshape(b, 3 * r, c)
    out = pl.pallas_call(
        functools.partial(_fps_body, npoint, r, c),
        grid=(b,),
        in_specs=[_batch_spec((n, 3)), _batch_spec((3 * r, c))],
        out_specs=_batch_spec((npoint, 1)),
        out_shape=jax.ShapeDtypeStruct((b, npoint, 1), jnp.int32),
    )(xyz, xyz_d)
    return out


def _knn_body(k, has_qidx, global_ofs, *refs):
    if has_qidx:
        xyzR_ref, xyzT_ref, qidx_ref, out_ref = refs
    else:
        xyzR_ref, xyzT_ref, out_ref = refs
    n = xyzT_ref.shape[-1]
    if has_qidx:
        nq = qidx_ref.shape[0]
        lane_q = jax.lax.broadcasted_iota(jnp.int32, (nq, n), 1)
        oh = (lane_q == qidx_ref[:, :]).astype(jnp.float32)
        q = jnp.dot(oh, xyzR_ref[...], preferred_element_type=jnp.float32)
    else:
        nq = xyzR_ref.shape[0]
        q = xyzR_ref[...]
    qx = q[:, 0:1]
    qy = q[:, 1:2]
    qz = q[:, 2:3]
    dx = qx - xyzT_ref[0:1, :]
    dy = qy - xyzT_ref[1:2, :]
    dz = qz - xyzT_ref[2:3, :]
    d = dx * dx + dy * dy + dz * dz
    lane = jax.lax.broadcasted_iota(jnp.int32, (nq, n), 1)
    big = jnp.float32(np.inf)
    ofs = pl.program_id(0) * n if global_ofs else 0
    for j in range(k):
        m = jnp.min(d, axis=1, keepdims=True)
        sel = jnp.where(d == m, lane, n)
        amin = jnp.min(sel, axis=1, keepdims=True)
        out_ref[:, pl.ds(j, 1)] = amin + ofs
        d = jnp.where(lane == amin, big, d)


def _knn_self(xyz, k, global_ofs=False):
    b, n, _ = xyz.shape
    xyz_t = jnp.transpose(xyz, (0, 2, 1))
    return pl.pallas_call(
        functools.partial(_knn_body, k, False, global_ofs),
        grid=(b,),
        in_specs=[_batch_spec((n, 3)), _batch_spec((3, n))],
        out_specs=_batch_spec((n, k)),
        out_shape=jax.ShapeDtypeStruct((b, n, k), jnp.int32),
    )(xyz, xyz_t)


def _knn_fps(xyz, qidx, k):
    b, n, _ = xyz.shape
    nq = qidx.shape[1]
    xyz_t = jnp.transpose(xyz, (0, 2, 1))
    return pl.pallas_call(
        functools.partial(_knn_body, k, True, False),
        grid=(b,),
        in_specs=[_batch_spec((n, 3)), _batch_spec((3, n)),
                  _batch_spec((nq, 1))],
        out_specs=_batch_spec((nq, k)),
        out_shape=jax.ShapeDtypeStruct((b, nq, k), jnp.int32),
    )(xyz, xyz_t, qidx)


def _tb_pre_body(xyzR_ref, f_ref, fc1w_ref, fc1b_ref, wq_ref, wk_ref,
                 wv_ref, d1w_ref, q_ref, t_ref):
    x = jnp.dot(f_ref[...], fc1w_ref[...],
                preferred_element_type=jnp.float32) + fc1b_ref[...]
    q_ref[...] = jnp.dot(x, wq_ref[...], preferred_element_type=jnp.float32)
    t_ref[:, 0:128] = jnp.dot(x, wk_ref[...],
                              preferred_element_type=jnp.float32)
    t_ref[:, 128:256] = jnp.dot(x, wv_ref[...],
                                preferred_element_type=jnp.float32)
    t_ref[:, 256:384] = jnp.dot(xyzR_ref[...], d1w_ref[...],
                                preferred_element_type=jnp.float32)


def _tb_post_body(k, tile, d1b_ref, d2w_ref, d2b_ref, g1w_ref, g1b_ref,
                  g2w_ref, g2b_ref, fc2w_ref, fc2b_ref, t_ref, q_ref,
                  knn_ref, pre_ref, out_ref, a_sc, w_sc):
    n = t_ref.shape[0]
    tid = pl.program_id(1)
    pq = t_ref[pl.ds(tid * tile, tile), 256:384]
    qv = q_ref[...]
    knn = knn_ref[...]
    lane = jax.lax.broadcasted_iota(jnp.int32, (tile, n), 1)
    table = t_ref[...]
    for j in range(k):
        idx = knn[:, j:j + 1]
        oh = (lane == idx).astype(jnp.float32)
        g = jnp.dot(oh, table, preferred_element_type=jnp.float32)
        xk = g[:, 0:128]
        xv = g[:, 128:256]
        pg = g[:, 256:384]
        pos = jnp.maximum(pq - pg + d1b_ref[...], 0.0)
        pos = jnp.dot(pos, d2w_ref[...],
                      preferred_element_type=jnp.float32) + d2b_ref[...]
        u = qv - xk + pos
        a = jnp.maximum(jnp.dot(u, g1w_ref[...],
                                preferred_element_type=jnp.float32)
                        + g1b_ref[...], 0.0)
        a = jnp.dot(a, g2w_ref[...],
                    preferred_element_type=jnp.float32) + g2b_ref[...]
        a_sc[j] = a / _SQRT_DM
        w_sc[j] = xv + pos
    m = a_sc[0]
    for j in range(1, k):
        m = jnp.maximum(m, a_sc[j])
    s = jnp.zeros((tile, _DM), jnp.float32)
    acc = jnp.zeros((tile, _DM), jnp.float32)
    for j in range(k):
        e = jnp.exp(a_sc[j] - m)
        s = s + e
        acc = acc + e * w_sc[j]
    res = acc / s
    out_ref[...] = (jnp.dot(res, fc2w_ref[...],
                            preferred_element_type=jnp.float32)
                    + fc2b_ref[...] + pre_ref[...])


def _tb_post_g_body(k, tile, d1b_ref, d2w_ref, d2b_ref, g1w_ref, g1b_ref,
                    g2w_ref, g2b_ref, fc2w_ref, fc2b_ref, g_ref, tq_ref,
                    q_ref, pre_ref, out_ref, a_sc, w_sc):
    pq = tq_ref[:, 256:384]
    qv = q_ref[...]
    for j in range(k):
        base = j * 384
        xk = g_ref[:, base:base + 128]
        xv = g_ref[:, base + 128:base + 256]
        pg = g_ref[:, base + 256:base + 384]
        pos = jnp.maximum(pq - pg + d1b_ref[...], 0.0)
        pos = jnp.dot(pos, d2w_ref[...],
                      preferred_element_type=jnp.float32) + d2b_ref[...]
        u = qv - xk + pos
        a = jnp.maximum(jnp.dot(u, g1w_ref[...],
                                preferred_element_type=jnp.float32)
                        + g1b_ref[...], 0.0)
        a = jnp.dot(a, g2w_ref[...],
                    preferred_element_type=jnp.float32) + g2b_ref[...]
        a_sc[j] = a / _SQRT_DM
        w_sc[j] = xv + pos
    m = a_sc[0]
    for j in range(1, k):
        m = jnp.maximum(m, a_sc[j])
    s = jnp.zeros((tile, _DM), jnp.float32)
    acc = jnp.zeros((tile, _DM), jnp.float32)
    for j in range(k):
        e = jnp.exp(a_sc[j] - m)
        s = s + e
        acc = acc + e * w_sc[j]
    res = acc / s
    out_ref[...] = (jnp.dot(res, fc2w_ref[...],
                            preferred_element_type=jnp.float32)
                    + fc2b_ref[...] + pre_ref[...])


def _tb(p, xyz, feats, knn, use_sc=False):
    b, n, d_in = feats.shape
    k = knn.shape[2]
    fc1w, fc1b = p['fc1']
    d1w, d1b = p['d1']
    d2w, d2b = p['d2']
    g1w, g1b = p['g1']
    g2w, g2b = p['g2']
    fc2w, fc2b = p['fc2']
    q, t = pl.pallas_call(
        _tb_pre_body,
        grid=(b,),
        in_specs=[_batch_spec((n, 3)), _batch_spec((n, d_in)),
                  _rep_spec(fc1w.shape), _rep_spec((1, _DM)),
                  _rep_spec(p['wq'].shape), _rep_spec(p['wk'].shape),
                  _rep_spec(p['wv'].shape), _rep_spec(d1w.shape)],
        out_specs=[_batch_spec((n, _DM)), _batch_spec((n, 384))],
        out_shape=[jax.ShapeDtypeStruct((b, n, _DM), jnp.float32),
                   jax.ShapeDtypeStruct((b, n, 384), jnp.float32)],
    )(xyz, feats, fc1w, fc1b.reshape(1, -1), p['wq'], p['wk'], p['wv'], d1w)

    if use_sc:
        chunk = _sc_chunk(b * n * k)
        g = _sc_gather(t.reshape(b * n, 384), knn.reshape(-1), chunk)
        gr = g.reshape(b, n, k * 384)
        tile = min(n, 128)
        nt = n // tile
        out = pl.pallas_call(
            functools.partial(_tb_post_g_body, k, tile),
            grid=(b, nt),
            in_specs=[_rep_spec((1, _DM)), _rep_spec(d2w.shape),
                      _rep_spec((1, _DM)), _rep_spec(g1w.shape),
                      _rep_spec((1, _DM)), _rep_spec(g2w.shape),
                      _rep_spec((1, _DM)), _rep_spec(fc2w.shape),
                      _rep_spec((1, d_in)),
                      pl.BlockSpec((None, tile, k * 384),
                                   lambda b_, t_: (b_, t_, 0)),
                      pl.BlockSpec((None, tile, 384),
                                   lambda b_, t_: (b_, t_, 0)),
                      pl.BlockSpec((None, tile, _DM),
                                   lambda b_, t_: (b_, t_, 0)),
                      pl.BlockSpec((None, tile, d_in),
                                   lambda b_, t_: (b_, t_, 0))],
            out_specs=pl.BlockSpec((None, tile, d_in),
                                   lambda b_, t_: (b_, t_, 0)),
            out_shape=jax.ShapeDtypeStruct((b, n, d_in), jnp.float32),
            scratch_shapes=[pltpu.VMEM((k, tile, _DM), jnp.float32),
                            pltpu.VMEM((k, tile, _DM), jnp.float32)],
        )(d1b.reshape(1, -1), d2w, d2b.reshape(1, -1), g1w,
          g1b.reshape(1, -1), g2w, g2b.reshape(1, -1), fc2w,
          fc2b.reshape(1, -1), gr, t, q, feats)
        return out

    tile = min(n, 256)
    nt = n // tile
    out = pl.pallas_call(
        functools.partial(_tb_post_body, k, tile),
        grid=(b, nt),
        in_specs=[_rep_spec((1, _DM)), _rep_spec(d2w.shape),
                  _rep_spec((1, _DM)), _rep_spec(g1w.shape),
                  _rep_spec((1, _DM)), _rep_spec(g2w.shape),
                  _rep_spec((1, _DM)), _rep_spec(fc2w.shape),
                  _rep_spec((1, d_in)),
                  pl.BlockSpec((None, n, 384), lambda b_, t_: (b_, 0, 0)),
                  pl.BlockSpec((None, tile, _DM), lambda b_, t_: (b_, t_, 0)),
                  pl.BlockSpec((None, tile, k), lambda b_, t_: (b_, t_, 0)),
                  pl.BlockSpec((None, tile, d_in), lambda b_, t_: (b_, t_, 0))],
        out_specs=pl.BlockSpec((None, tile, d_in), lambda b_, t_: (b_, t_, 0)),
        out_shape=jax.ShapeDtypeStruct((b, n, d_in), jnp.float32),
        scratch_shapes=[pltpu.VMEM((k, tile, _DM), jnp.float32),
                        pltpu.VMEM((k, tile, _DM), jnp.float32)],
    )(d1b.reshape(1, -1), d2w, d2b.reshape(1, -1), g1w, g1b.reshape(1, -1),
      g2w, g2b.reshape(1, -1), fc2w, fc2b.reshape(1, -1), t, q, knn, feats)
    return out


def _td_body(k, xyzR_ref, f_ref, fps_ref, knn_ref, l1wx_ref, l1wf_ref,
             l1b_ref, l2w_ref, l2b_ref, nxyz_ref, out_ref):
    n = xyzR_ref.shape[0]
    npt = fps_ref.shape[0]
    c_out = l2w_ref.shape[0]
    lane = jax.lax.broadcasted_iota(jnp.int32, (npt, n), 1)
    oh_fps = (lane == fps_ref[:, :]).astype(jnp.float32)
    new_xyz = jnp.dot(oh_fps, xyzR_ref[...],
                      preferred_element_type=jnp.float32)
    nxyz_ref[...] = new_xyz
    knn = knn_ref[...]
    m = jnp.full((npt, c_out), -jnp.inf, jnp.float32)
    for j in range(k):
        idx = knn[:, j:j + 1]
        oh = (lane == idx).astype(jnp.float32)
        gx = jnp.dot(oh, xyzR_ref[...],
                     preferred_element_type=jnp.float32) - new_xyz
        gf = jnp.dot(oh, f_ref[...], preferred_element_type=jnp.float32)
        h = (jnp.dot(gx, l1wx_ref[...], preferred_element_type=jnp.float32)
             + jnp.dot(gf, l1wf_ref[...], preferred_element_type=jnp.float32)
             + l1b_ref[...])
        h = jnp.maximum(h, 0.0)
        h = jnp.dot(h, l2w_ref[...],
                    preferred_element_type=jnp.float32) + l2b_ref[...]
        h = jnp.maximum(h, 0.0)
        m = jnp.maximum(m, h)
    out_ref[...] = m


def _td(p, xyz, feats, fps, knn):
    b, n, c_in = feats.shape
    npt = fps.shape[1]
    k = knn.shape[2]
    l1w, l1b = p['l1']
    l2w, l2b = p['l2']
    c_out = l2w.shape[1]
    nxyz, f_out = pl.pallas_call(
        functools.partial(_td_body, k),
        grid=(b,),
        in_specs=[_batch_spec((n, 3)), _batch_spec((n, c_in)),
                  _batch_spec((npt, 1)), _batch_spec((npt, k)),
                  _rep_spec((3, c_out)), _rep_spec((c_in, c_out)),
                  _rep_spec((1, c_out)), _rep_spec(l2w.shape),
                  _rep_spec((1, c_out))],
        out_specs=[_batch_spec((npt, 3)), _batch_spec((npt, c_out))],
        out_shape=[jax.ShapeDtypeStruct((b, npt, 3), jnp.float32),
                   jax.ShapeDtypeStruct((b, npt, c_out), jnp.float32)],
    )(xyz, feats, fps, knn, l1w[:3], l1w[3:], l1b.reshape(1, -1),
      l2w, l2b.reshape(1, -1))
    return nxyz, f_out


def _tu_body(fc_ref, xycR_ref, xycT_ref, ff_ref, xyf_ref, w1_ref, b1_ref,
             w2_ref, b2_ref, out_ref):
    nc = xycR_ref.shape[0]
    nf = xyf_ref.shape[0]
    f1 = jnp.maximum(jnp.dot(fc_ref[...], w1_ref[...],
                             preferred_element_type=jnp.float32)
                     + b1_ref[...], 0.0)
    f2 = jnp.maximum(jnp.dot(ff_ref[...], w2_ref[...],
                             preferred_element_type=jnp.float32)
                     + b2_ref[...], 0.0)
    dx = xyf_ref[:, 0:1] - xycT_ref[0:1, :]
    dy = xyf_ref[:, 1:2] - xycT_ref[1:2, :]
    dz = xyf_ref[:, 2:3] - xycT_ref[2:3, :]
    d = dx * dx + dy * dy + dz * dz
    lane = jax.lax.broadcasted_iota(jnp.int32, (nf, nc), 1)
    big = jnp.float32(np.inf)
    ws = []
    idxs = []
    for j in range(3):
        m = jnp.min(d, axis=1, keepdims=True)
        sel = jnp.where(d == m, lane, nc)
        amin = jnp.min(sel, axis=1, keepdims=True)
        ws.append(1.0 / jnp.maximum(m, 1e-10))
        idxs.append(amin)
        d = jnp.where(lane == amin, big, d)
    wsum = (ws[0] + ws[1]) + ws[2]
    acc = None
    for j in range(3):
        oh = (lane == idxs[j]).astype(jnp.float32)
        fj = jnp.dot(oh, f1, preferred_element_type=jnp.float32)
        term = (ws[j] / wsum) * fj
        acc = term if acc is None else acc + term
    out_ref[...] = acc + f2


def _tu(p, f_coarse, xyz_coarse, f_fine, xyz_fine):
    b, nc, _ = xyz_coarse.shape
    nf = xyz_fine.shape[1]
    w1, b1 = p['fc1']
    w2, b2 = p['fc2']
    d = w1.shape[1]
    xyc_t = jnp.transpose(xyz_coarse, (0, 2, 1))
    return pl.pallas_call(
        _tu_body,
        grid=(b,),
        in_specs=[_batch_spec(f_coarse.shape[1:]), _batch_spec((nc, 3)),
                  _batch_spec((3, nc)), _batch_spec(f_fine.shape[1:]),
                  _batch_spec((nf, 3)), _rep_spec(w1.shape),
                  _rep_spec((1, d)), _rep_spec(w2.shape), _rep_spec((1, d))],
        out_specs=_batch_spec((nf, d)),
        out_shape=jax.ShapeDtypeStruct((b, nf, d), jnp.float32),
    )(f_coarse, xyz_coarse, xyc_t, f_fine, xyz_fine, w1,
      b1.reshape(1, -1), w2, b2.reshape(1, -1))


def _mlp_body(relus, nlayer, *refs):
    x_ref = refs[0]
    out_ref = refs[-1]
    h = x_ref[...]
    for i in range(nlayer):
        w_ref = refs[1 + 2 * i]
        b_ref = refs[2 + 2 * i]
        h = jnp.dot(h, w_ref[...],
                    preferred_element_type=jnp.float32) + b_ref[...]
        if relus[i]:
            h = jnp.maximum(h, 0.0)
    out_ref[...] = h


def _mlp(x, layers, relus):
    b, n, _ = x.shape
    nlayer = len(layers)
    args = [x]
    specs = [_batch_spec(x.shape[1:])]
    for (w, bias) in layers:
        args.append(w)
        args.append(bias.reshape(1, -1))
        specs.append(_rep_spec(w.shape))
        specs.append(_rep_spec((1, w.shape[1])))
    d_out = layers[-1][0].shape[1]
    return pl.pallas_call(
        functools.partial(_mlp_body, relus, nlayer),
        grid=(b,),
        in_specs=specs,
        out_specs=_batch_spec((n, d_out)),
        out_shape=jax.ShapeDtypeStruct((b, n, d_out), jnp.float32),
    )(*args)


def _tb_stage(p, xyz, feats):
    n = xyz.shape[1]
    k = min(_KP, n)
    use_sc = False
    knn = _knn_self(xyz, k, global_ofs=use_sc)
    return _tb(p, xyz, feats, knn, use_sc=use_sc)


def kernel(x, params):
    xb = jnp.transpose(x, (0, 2, 1))
    xyz = xb
    f = _mlp(xb, [params['bb_fc1a'], params['bb_fc1b']], [True, False])
    f = _tb_stage(params['bb_tb0'], xyz, f)
    fac = [(f, xyz)]
    npts = xyz.shape[1]
    for i in range(4):
        npts //= 4
        fps = _fps(xyz, npts)
        knn_d = _knn_fps(xyz, fps, _KP)
        xyz, f = _td(params['bb_td'][i], xyz, f, fps, knn_d)
        f = _tb_stage(params['bb_tbs'][i], xyz, f)
        fac.append((f, xyz))
    feature, coord = fac[-1]
    h = _mlp(feature, [params['mlp2a'], params['mlp2b'], params['mlp2c']],
             [True, True, False])
    feature = _tb_stage(params['t2'], coord, h)
    for i in range(4):
        f_fine, c_fine = fac[-i - 2]
        feature = _tu(params['tu'][i], feature, coord, f_fine, c_fine)
        coord = c_fine
        feature = _tb_stage(params['tbu'][i], coord, feature)
    h = _mlp(feature, [params['mlp3a'], params['mlp3b'], params['mlp3c']],
             [True, True, False])
    return h


# speedup vs baseline: 1.3538x; 1.1171x over previous
import functools

import jax
import jax.numpy as jnp
import numpy as np
from jax.experimental import pallas as pl
from jax.experimental.pallas import tpu as pltpu
from jax.experimental.pallas import tpu_sc as plsc

_B = 2
_KP = 16
_DM = 128
_SQRT_DM = np.float32(np.sqrt(128.0))

_SC_NC = 2
_SC_NS = 16
_SC_NW = _SC_NC * _SC_NS


def _sc_gather(table, idx, chunk):
    bn = idx.shape[0]
    d = table.shape[1]
    per_w = bn // (chunk * _SC_NW)
    mesh = plsc.VectorSubcoreMesh(core_axis_name="c", subcore_axis_name="s",
                                  num_cores=_SC_NC, num_subcores=_SC_NS)

    def body(table_hbm, idx_hbm, out_hbm, idx_v, rows_v, sem):
        wid = jax.lax.axis_index("s") * _SC_NC + jax.lax.axis_index("c")
        for j in range(per_w):
            base = (wid * per_w + j) * chunk
            pltpu.sync_copy(idx_hbm.at[pl.ds(base, chunk)], idx_v)
            pltpu.async_copy(table_hbm.at[idx_v], rows_v, sem).wait()
            pltpu.sync_copy(rows_v, out_hbm.at[pl.ds(base, chunk)])

    f = pl.kernel(
        body,
        out_type=jax.ShapeDtypeStruct((bn, d), jnp.float32),
        mesh=mesh,
        scratch_types=[pltpu.VMEM((chunk,), jnp.int32),
                       pltpu.VMEM((chunk, d), jnp.float32),
                       pltpu.SemaphoreType.DMA],
    )
    return f(table, idx)


def _sc_chunk(bn):
    for c in (128, 64, 32, 16, 8):
        if bn % (c * _SC_NW) == 0:
            return c
    return 0


def _rep_spec(shape):
    nd = len(shape)
    return pl.BlockSpec(shape, lambda *_: (0,) * nd)


def _batch_spec(shape):
    nd = len(shape)
    return pl.BlockSpec((None,) + shape, lambda b: (b,) + (0,) * nd)


def _fps_body(npoint, bsz, xyzR_ref, xyzT_ref, out_ref):
    n = xyzT_ref.shape[-1]
    lane = jax.lax.broadcasted_iota(jnp.int32, (1, n), 1)
    xs = [xyzT_ref[b, 0:1, :] for b in range(bsz)]
    ys = [xyzT_ref[b, 1:2, :] for b in range(bsz)]
    zs = [xyzT_ref[b, 2:3, :] for b in range(bsz)]

    def body(i, carry):
        dists, fars = carry
        new_d, new_f = [], []
        for b in range(bsz):
            out_ref[b, pl.ds(i, 1), :] = jnp.reshape(fars[b], (1, 1))
            row = xyzR_ref[b, pl.ds(fars[b], 1), :]
            dx = xs[b] - row[:, 0:1]
            dy = ys[b] - row[:, 1:2]
            dz = zs[b] - row[:, 2:3]
            dist = dx * dx + dy * dy + dz * dz
            dist_min = jnp.minimum(dists[b], dist)
            m = jnp.max(dist_min)
            sel = jnp.where(dist_min == m, lane, n)
            new_d.append(dist_min)
            new_f.append(jnp.min(sel))
        return tuple(new_d), tuple(new_f)

    init = (tuple(jnp.full((1, n), 1e10, dtype=jnp.float32)
                  for _ in range(bsz)),
            tuple(jnp.int32(0) for _ in range(bsz)))
    jax.lax.fori_loop(0, npoint, body, init)


def _fps(xyz, npoint):
    b, n, _ = xyz.shape
    xyz_t = jnp.transpose(xyz, (0, 2, 1))
    out = pl.pallas_call(
        functools.partial(_fps_body, npoint, b),
        grid=(1,),
        in_specs=[_rep_spec((b, n, 3)), _rep_spec((b, 3, n))],
        out_specs=_rep_spec((b, npoint, 1)),
        out_shape=jax.ShapeDtypeStruct((b, npoint, 1), jnp.int32),
    )(xyz, xyz_t)
    return out


def _knn_body(k, has_qidx, global_ofs, *refs):
    if has_qidx:
        xyzR_ref, xyzT_ref, qidx_ref, out_ref = refs
    else:
        xyzR_ref, xyzT_ref, out_ref = refs
    n = xyzT_ref.shape[-1]
    if has_qidx:
        nq = qidx_ref.shape[0]
        lane_q = jax.lax.broadcasted_iota(jnp.int32, (nq, n), 1)
        oh = (lane_q == qidx_ref[:, :]).astype(jnp.float32)
        q = jnp.dot(oh, xyzR_ref[...], preferred_element_type=jnp.float32)
    else:
        nq = xyzR_ref.shape[0]
        q = xyzR_ref[...]
    qx = q[:, 0:1]
    qy = q[:, 1:2]
    qz = q[:, 2:3]
    dx = qx - xyzT_ref[0:1, :]
    dy = qy - xyzT_ref[1:2, :]
    dz = qz - xyzT_ref[2:3, :]
    d = dx * dx + dy * dy + dz * dz
    lane = jax.lax.broadcasted_iota(jnp.int32, (nq, n), 1)
    big = jnp.float32(np.inf)
    ofs = pl.program_id(0) * n if global_ofs else 0
    for j in range(k):
        m = jnp.min(d, axis=1, keepdims=True)
        sel = jnp.where(d == m, lane, n)
        amin = jnp.min(sel, axis=1, keepdims=True)
        out_ref[:, pl.ds(j, 1)] = amin + ofs
        d = jnp.where(lane == amin, big, d)


def _knn_self(xyz, k, global_ofs=False):
    b, n, _ = xyz.shape
    xyz_t = jnp.transpose(xyz, (0, 2, 1))
    return pl.pallas_call(
        functools.partial(_knn_body, k, False, global_ofs),
        grid=(b,),
        in_specs=[_batch_spec((n, 3)), _batch_spec((3, n))],
        out_specs=_batch_spec((n, k)),
        out_shape=jax.ShapeDtypeStruct((b, n, k), jnp.int32),
    )(xyz, xyz_t)


def _knn_fps(xyz, qidx, k):
    b, n, _ = xyz.shape
    nq = qidx.shape[1]
    xyz_t = jnp.transpose(xyz, (0, 2, 1))
    return pl.pallas_call(
        functools.partial(_knn_body, k, True, False),
        grid=(b,),
        in_specs=[_batch_spec((n, 3)), _batch_spec((3, n)),
                  _batch_spec((nq, 1))],
        out_specs=_batch_spec((nq, k)),
        out_shape=jax.ShapeDtypeStruct((b, nq, k), jnp.int32),
    )(xyz, xyz_t, qidx)


def _tb_pre_body(xyzR_ref, f_ref, fc1w_ref, fc1b_ref, wq_ref, wk_ref,
                 wv_ref, d1w_ref, q_ref, t_ref):
    x = jnp.dot(f_ref[...], fc1w_ref[...],
                preferred_element_type=jnp.float32) + fc1b_ref[...]
    q_ref[...] = jnp.dot(x, wq_ref[...], preferred_element_type=jnp.float32)
    t_ref[:, 0:128] = jnp.dot(x, wk_ref[...],
                              preferred_element_type=jnp.float32)
    t_ref[:, 128:256] = jnp.dot(x, wv_ref[...],
                                preferred_element_type=jnp.float32)
    t_ref[:, 256:384] = jnp.dot(xyzR_ref[...], d1w_ref[...],
                                preferred_element_type=jnp.float32)


def _tb_post_body(k, tile, d1b_ref, d2w_ref, d2b_ref, g1w_ref, g1b_ref,
                  g2w_ref, g2b_ref, fc2w_ref, fc2b_ref, t_ref, q_ref,
                  knn_ref, pre_ref, out_ref, a_sc, w_sc):
    n = t_ref.shape[0]
    tid = pl.program_id(1)
    pq = t_ref[pl.ds(tid * tile, tile), 256:384]
    qv = q_ref[...]
    knn = knn_ref[...]
    lane = jax.lax.broadcasted_iota(jnp.int32, (tile, n), 1)
    table = t_ref[...]
    for j in range(k):
        idx = knn[:, j:j + 1]
        oh = (lane == idx).astype(jnp.float32)
        g = jnp.dot(oh, table, preferred_element_type=jnp.float32)
        xk = g[:, 0:128]
        xv = g[:, 128:256]
        pg = g[:, 256:384]
        pos = jnp.maximum(pq - pg + d1b_ref[...], 0.0)
        pos = jnp.dot(pos, d2w_ref[...],
                      preferred_element_type=jnp.float32) + d2b_ref[...]
        u = qv - xk + pos
        a = jnp.maximum(jnp.dot(u, g1w_ref[...],
                                preferred_element_type=jnp.float32)
                        + g1b_ref[...], 0.0)
        a = jnp.dot(a, g2w_ref[...],
                    preferred_element_type=jnp.float32) + g2b_ref[...]
        a_sc[j] = a / _SQRT_DM
        w_sc[j] = xv + pos
    m = a_sc[0]
    for j in range(1, k):
        m = jnp.maximum(m, a_sc[j])
    s = jnp.zeros((tile, _DM), jnp.float32)
    acc = jnp.zeros((tile, _DM), jnp.float32)
    for j in range(k):
        e = jnp.exp(a_sc[j] - m)
        s = s + e
        acc = acc + e * w_sc[j]
    res = acc / s
    out_ref[...] = (jnp.dot(res, fc2w_ref[...],
                            preferred_element_type=jnp.float32)
                    + fc2b_ref[...] + pre_ref[...])


def _tb_post_g_body(k, tile, d1b_ref, d2w_ref, d2b_ref, g1w_ref, g1b_ref,
                    g2w_ref, g2b_ref, fc2w_ref, fc2b_ref, g_ref, tq_ref,
                    q_ref, pre_ref, out_ref, a_sc, w_sc):
    pq = tq_ref[:, 256:384]
    qv = q_ref[...]
    for j in range(k):
        base = j * 384
        xk = g_ref[:, base:base + 128]
        xv = g_ref[:, base + 128:base + 256]
        pg = g_ref[:, base + 256:base + 384]
        pos = jnp.maximum(pq - pg + d1b_ref[...], 0.0)
        pos = jnp.dot(pos, d2w_ref[...],
                      preferred_element_type=jnp.float32) + d2b_ref[...]
        u = qv - xk + pos
        a = jnp.maximum(jnp.dot(u, g1w_ref[...],
                                preferred_element_type=jnp.float32)
                        + g1b_ref[...], 0.0)
        a = jnp.dot(a, g2w_ref[...],
                    preferred_element_type=jnp.float32) + g2b_ref[...]
        a_sc[j] = a / _SQRT_DM
        w_sc[j] = xv + pos
    m = a_sc[0]
    for j in range(1, k):
        m = jnp.maximum(m, a_sc[j])
    s = jnp.zeros((tile, _DM), jnp.float32)
    acc = jnp.zeros((tile, _DM), jnp.float32)
    for j in range(k):
        e = jnp.exp(a_sc[j] - m)
        s = s + e
        acc = acc + e * w_sc[j]
    res = acc / s
    out_ref[...] = (jnp.dot(res, fc2w_ref[...],
                            preferred_element_type=jnp.float32)
                    + fc2b_ref[...] + pre_ref[...])


def _tb(p, xyz, feats, knn, use_sc=False):
    b, n, d_in = feats.shape
    k = knn.shape[2]
    fc1w, fc1b = p['fc1']
    d1w, d1b = p['d1']
    d2w, d2b = p['d2']
    g1w, g1b = p['g1']
    g2w, g2b = p['g2']
    fc2w, fc2b = p['fc2']
    q, t = pl.pallas_call(
        _tb_pre_body,
        grid=(b,),
        in_specs=[_batch_spec((n, 3)), _batch_spec((n, d_in)),
                  _rep_spec(fc1w.shape), _rep_spec((1, _DM)),
                  _rep_spec(p['wq'].shape), _rep_spec(p['wk'].shape),
                  _rep_spec(p['wv'].shape), _rep_spec(d1w.shape)],
        out_specs=[_batch_spec((n, _DM)), _batch_spec((n, 384))],
        out_shape=[jax.ShapeDtypeStruct((b, n, _DM), jnp.float32),
                   jax.ShapeDtypeStruct((b, n, 384), jnp.float32)],
    )(xyz, feats, fc1w, fc1b.reshape(1, -1), p['wq'], p['wk'], p['wv'], d1w)

    if use_sc:
        chunk = _sc_chunk(b * n * k)
        g = _sc_gather(t.reshape(b * n, 384), knn.reshape(-1), chunk)
        gr = g.reshape(b, n, k * 384)
        tile = min(n, 128)
        nt = n // tile
        out = pl.pallas_call(
            functools.partial(_tb_post_g_body, k, tile),
            grid=(b, nt),
            in_specs=[_rep_spec((1, _DM)), _rep_spec(d2w.shape),
                      _rep_spec((1, _DM)), _rep_spec(g1w.shape),
                      _rep_spec((1, _DM)), _rep_spec(g2w.shape),
                      _rep_spec((1, _DM)), _rep_spec(fc2w.shape),
                      _rep_spec((1, d_in)),
                      pl.BlockSpec((None, tile, k * 384),
                                   lambda b_, t_: (b_, t_, 0)),
                      pl.BlockSpec((None, tile, 384),
                                   lambda b_, t_: (b_, t_, 0)),
                      pl.BlockSpec((None, tile, _DM),
                                   lambda b_, t_: (b_, t_, 0)),
                      pl.BlockSpec((None, tile, d_in),
                                   lambda b_, t_: (b_, t_, 0))],
            out_specs=pl.BlockSpec((None, tile, d_in),
                                   lambda b_, t_: (b_, t_, 0)),
            out_shape=jax.ShapeDtypeStruct((b, n, d_in), jnp.float32),
            scratch_shapes=[pltpu.VMEM((k, tile, _DM), jnp.float32),
                            pltpu.VMEM((k, tile, _DM), jnp.float32)],
        )(d1b.reshape(1, -1), d2w, d2b.reshape(1, -1), g1w,
          g1b.reshape(1, -1), g2w, g2b.reshape(1, -1), fc2w,
          fc2b.reshape(1, -1), gr, t, q, feats)
        return out

    tile = min(n, 256)
    nt = n // tile
    out = pl.pallas_call(
        functools.partial(_tb_post_body, k, tile),
        grid=(b, nt),
        in_specs=[_rep_spec((1, _DM)), _rep_spec(d2w.shape),
                  _rep_spec((1, _DM)), _rep_spec(g1w.shape),
                  _rep_spec((1, _DM)), _rep_spec(g2w.shape),
                  _rep_spec((1, _DM)), _rep_spec(fc2w.shape),
                  _rep_spec((1, d_in)),
                  pl.BlockSpec((None, n, 384), lambda b_, t_: (b_, 0, 0)),
                  pl.BlockSpec((None, tile, _DM), lambda b_, t_: (b_, t_, 0)),
                  pl.BlockSpec((None, tile, k), lambda b_, t_: (b_, t_, 0)),
                  pl.BlockSpec((None, tile, d_in), lambda b_, t_: (b_, t_, 0))],
        out_specs=pl.BlockSpec((None, tile, d_in), lambda b_, t_: (b_, t_, 0)),
        out_shape=jax.ShapeDtypeStruct((b, n, d_in), jnp.float32),
        scratch_shapes=[pltpu.VMEM((k, tile, _DM), jnp.float32),
                        pltpu.VMEM((k, tile, _DM), jnp.float32)],
    )(d1b.reshape(1, -1), d2w, d2b.reshape(1, -1), g1w, g1b.reshape(1, -1),
      g2w, g2b.reshape(1, -1), fc2w, fc2b.reshape(1, -1), t, q, knn, feats)
    return out


def _td_body(k, xyzR_ref, f_ref, fps_ref, knn_ref, l1wx_ref, l1wf_ref,
             l1b_ref, l2w_ref, l2b_ref, nxyz_ref, out_ref):
    n = xyzR_ref.shape[0]
    npt = fps_ref.shape[0]
    c_out = l2w_ref.shape[0]
    lane = jax.lax.broadcasted_iota(jnp.int32, (npt, n), 1)
    oh_fps = (lane == fps_ref[:, :]).astype(jnp.float32)
    new_xyz = jnp.dot(oh_fps, xyzR_ref[...],
                      preferred_element_type=jnp.float32)
    nxyz_ref[...] = new_xyz
    knn = knn_ref[...]
    m = jnp.full((npt, c_out), -jnp.inf, jnp.float32)
    for j in range(k):
        idx = knn[:, j:j + 1]
        oh = (lane == idx).astype(jnp.float32)
        gx = jnp.dot(oh, xyzR_ref[...],
                     preferred_element_type=jnp.float32) - new_xyz
        gf = jnp.dot(oh, f_ref[...], preferred_element_type=jnp.float32)
        h = (jnp.dot(gx, l1wx_ref[...], preferred_element_type=jnp.float32)
             + jnp.dot(gf, l1wf_ref[...], preferred_element_type=jnp.float32)
             + l1b_ref[...])
        h = jnp.maximum(h, 0.0)
        h = jnp.dot(h, l2w_ref[...],
                    preferred_element_type=jnp.float32) + l2b_ref[...]
        h = jnp.maximum(h, 0.0)
        m = jnp.maximum(m, h)
    out_ref[...] = m


def _td(p, xyz, feats, fps, knn):
    b, n, c_in = feats.shape
    npt = fps.shape[1]
    k = knn.shape[2]
    l1w, l1b = p['l1']
    l2w, l2b = p['l2']
    c_out = l2w.shape[1]
    nxyz, f_out = pl.pallas_call(
        functools.partial(_td_body, k),
        grid=(b,),
        in_specs=[_batch_spec((n, 3)), _batch_spec((n, c_in)),
                  _batch_spec((npt, 1)), _batch_spec((npt, k)),
                  _rep_spec((3, c_out)), _rep_spec((c_in, c_out)),
                  _rep_spec((1, c_out)), _rep_spec(l2w.shape),
                  _rep_spec((1, c_out))],
        out_specs=[_batch_spec((npt, 3)), _batch_spec((npt, c_out))],
        out_shape=[jax.ShapeDtypeStruct((b, npt, 3), jnp.float32),
                   jax.ShapeDtypeStruct((b, npt, c_out), jnp.float32)],
    )(xyz, feats, fps, knn, l1w[:3], l1w[3:], l1b.reshape(1, -1),
      l2w, l2b.reshape(1, -1))
    return nxyz, f_out


def _tu_body(fc_ref, xycR_ref, xycT_ref, ff_ref, xyf_ref, w1_ref, b1_ref,
             w2_ref, b2_ref, out_ref):
    nc = xycR_ref.shape[0]
    nf = xyf_ref.shape[0]
    f1 = jnp.maximum(jnp.dot(fc_ref[...], w1_ref[...],
                             preferred_element_type=jnp.float32)
                     + b1_ref[...], 0.0)
    f2 = jnp.maximum(jnp.dot(ff_ref[...], w2_ref[...],
                             preferred_element_type=jnp.float32)
                     + b2_ref[...], 0.0)
    dx = xyf_ref[:, 0:1] - xycT_ref[0:1, :]
    dy = xyf_ref[:, 1:2] - xycT_ref[1:2, :]
    dz = xyf_ref[:, 2:3] - xycT_ref[2:3, :]
    d = dx * dx + dy * dy + dz * dz
    lane = jax.lax.broadcasted_iota(jnp.int32, (nf, nc), 1)
    big = jnp.float32(np.inf)
    ws = []
    idxs = []
    for j in range(3):
        m = jnp.min(d, axis=1, keepdims=True)
        sel = jnp.where(d == m, lane, nc)
        amin = jnp.min(sel, axis=1, keepdims=True)
        ws.append(1.0 / jnp.maximum(m, 1e-10))
        idxs.append(amin)
        d = jnp.where(lane == amin, big, d)
    wsum = (ws[0] + ws[1]) + ws[2]
    acc = None
    for j in range(3):
        oh = (lane == idxs[j]).astype(jnp.float32)
        fj = jnp.dot(oh, f1, preferred_element_type=jnp.float32)
        term = (ws[j] / wsum) * fj
        acc = term if acc is None else acc + term
    out_ref[...] = acc + f2


def _tu(p, f_coarse, xyz_coarse, f_fine, xyz_fine):
    b, nc, _ = xyz_coarse.shape
    nf = xyz_fine.shape[1]
    w1, b1 = p['fc1']
    w2, b2 = p['fc2']
    d = w1.shape[1]
    xyc_t = jnp.transpose(xyz_coarse, (0, 2, 1))
    return pl.pallas_call(
        _tu_body,
        grid=(b,),
        in_specs=[_batch_spec(f_coarse.shape[1:]), _batch_spec((nc, 3)),
                  _batch_spec((3, nc)), _batch_spec(f_fine.shape[1:]),
                  _batch_spec((nf, 3)), _rep_spec(w1.shape),
                  _rep_spec((1, d)), _rep_spec(w2.shape), _rep_spec((1, d))],
        out_specs=_batch_spec((nf, d)),
        out_shape=jax.ShapeDtypeStruct((b, nf, d), jnp.float32),
    )(f_coarse, xyz_coarse, xyc_t, f_fine, xyz_fine, w1,
      b1.reshape(1, -1), w2, b2.reshape(1, -1))


def _mlp_body(relus, nlayer, *refs):
    x_ref = refs[0]
    out_ref = refs[-1]
    h = x_ref[...]
    for i in range(nlayer):
        w_ref = refs[1 + 2 * i]
        b_ref = refs[2 + 2 * i]
        h = jnp.dot(h, w_ref[...],
                    preferred_element_type=jnp.float32) + b_ref[...]
        if relus[i]:
            h = jnp.maximum(h, 0.0)
    out_ref[...] = h


def _mlp(x, layers, relus):
    b, n, _ = x.shape
    nlayer = len(layers)
    args = [x]
    specs = [_batch_spec(x.shape[1:])]
    for (w, bias) in layers:
        args.append(w)
        args.append(bias.reshape(1, -1))
        specs.append(_rep_spec(w.shape))
        specs.append(_rep_spec((1, w.shape[1])))
    d_out = layers[-1][0].shape[1]
    return pl.pallas_call(
        functools.partial(_mlp_body, relus, nlayer),
        grid=(b,),
        in_specs=specs,
        out_specs=_batch_spec((n, d_out)),
        out_shape=jax.ShapeDtypeStruct((b, n, d_out), jnp.float32),
    )(*args)


def _tb_stage(p, xyz, feats):
    n = xyz.shape[1]
    k = min(_KP, n)
    use_sc = False
    knn = _knn_self(xyz, k, global_ofs=use_sc)
    return _tb(p, xyz, feats, knn, use_sc=use_sc)


def kernel(x, params):
    xb = jnp.transpose(x, (0, 2, 1))
    xyz = xb
    f = _mlp(xb, [params['bb_fc1a'], params['bb_fc1b']], [True, False])
    f = _tb_stage(params['bb_tb0'], xyz, f)
    fac = [(f, xyz)]
    npts = xyz.shape[1]
    for i in range(4):
        npts //= 4
        fps = _fps(xyz, npts)
        knn_d = _knn_fps(xyz, fps, _KP)
        xyz, f = _td(params['bb_td'][i], xyz, f, fps, knn_d)
        f = _tb_stage(params['bb_tbs'][i], xyz, f)
        fac.append((f, xyz))
    feature, coord = fac[-1]
    h = _mlp(feature, [params['mlp2a'], params['mlp2b'], params['mlp2c']],
             [True, True, False])
    feature = _tb_stage(params['t2'], coord, h)
    for i in range(4):
        f_fine, c_fine = fac[-i - 2]
        feature = _tu(params['tu'][i], feature, coord, f_fine, c_fine)
        coord = c_fine
        feature = _tb_stage(params['tbu'][i], coord, feature)
    h = _mlp(feature, [params['mlp3a'], params['mlp3b'], params['mlp3c']],
             [True, True, False])
    return h


# speedup vs baseline: 1.5395x; 1.1372x over previous
import functools

import jax
import jax.numpy as jnp
import numpy as np
from jax.experimental import pallas as pl
from jax.experimental.pallas import tpu as pltpu
from jax.experimental.pallas import tpu_sc as plsc

_B = 2
_KP = 16
_DM = 128
_SQRT_DM = np.float32(np.sqrt(128.0))

_SC_NC = 2
_SC_NS = 16
_SC_NW = _SC_NC * _SC_NS


def _sc_gather(table, idx, chunk):
    bn = idx.shape[0]
    d = table.shape[1]
    per_w = bn // (chunk * _SC_NW)
    mesh = plsc.VectorSubcoreMesh(core_axis_name="c", subcore_axis_name="s",
                                  num_cores=_SC_NC, num_subcores=_SC_NS)

    def body(table_hbm, idx_hbm, out_hbm, idx_v, rows_v, sem):
        wid = jax.lax.axis_index("s") * _SC_NC + jax.lax.axis_index("c")
        for j in range(per_w):
            base = (wid * per_w + j) * chunk
            pltpu.sync_copy(idx_hbm.at[pl.ds(base, chunk)], idx_v)
            pltpu.async_copy(table_hbm.at[idx_v], rows_v, sem).wait()
            pltpu.sync_copy(rows_v, out_hbm.at[pl.ds(base, chunk)])

    f = pl.kernel(
        body,
        out_type=jax.ShapeDtypeStruct((bn, d), jnp.float32),
        mesh=mesh,
        scratch_types=[pltpu.VMEM((chunk,), jnp.int32),
                       pltpu.VMEM((chunk, d), jnp.float32),
                       pltpu.SemaphoreType.DMA],
    )
    return f(table, idx)


def _sc_chunk(bn):
    for c in (128, 64, 32, 16, 8):
        if bn % (c * _SC_NW) == 0:
            return c
    return 0


def _rep_spec(shape):
    nd = len(shape)
    return pl.BlockSpec(shape, lambda *_: (0,) * nd)


def _batch_spec(shape):
    nd = len(shape)
    return pl.BlockSpec((None,) + shape, lambda b: (b,) + (0,) * nd)


def _fps_body(npoint, bsz, xyzR_ref, xyzT_ref, out_ref):
    n = xyzT_ref.shape[-1]
    lane = jax.lax.broadcasted_iota(jnp.int32, (1, n), 1)
    xs = [xyzT_ref[b, 0:1, :] for b in range(bsz)]
    ys = [xyzT_ref[b, 1:2, :] for b in range(bsz)]
    zs = [xyzT_ref[b, 2:3, :] for b in range(bsz)]

    ninf = jnp.float32(-np.inf)

    def body(i, carry):
        dists, fars = carry
        new_d, new_f = [], []
        for b in range(bsz):
            out_ref[b, pl.ds(i, 1), :] = fars[b]
            sel_mask = lane == fars[b]
            cx = jnp.max(jnp.where(sel_mask, xs[b], ninf),
                         axis=1, keepdims=True)
            cy = jnp.max(jnp.where(sel_mask, ys[b], ninf),
                         axis=1, keepdims=True)
            cz = jnp.max(jnp.where(sel_mask, zs[b], ninf),
                         axis=1, keepdims=True)
            dx = xs[b] - cx
            dy = ys[b] - cy
            dz = zs[b] - cz
            dist = dx * dx + dy * dy + dz * dz
            dist_min = jnp.minimum(dists[b], dist)
            m = jnp.max(dist_min, axis=1, keepdims=True)
            sel = jnp.where(dist_min == m, lane, n)
            new_d.append(dist_min)
            new_f.append(jnp.min(sel, axis=1, keepdims=True))
        return tuple(new_d), tuple(new_f)

    init = (tuple(jnp.full((1, n), 1e10, dtype=jnp.float32)
                  for _ in range(bsz)),
            tuple(jnp.zeros((1, 1), dtype=jnp.int32) for _ in range(bsz)))
    jax.lax.fori_loop(0, npoint, body, init)


def _fps(xyz, npoint):
    b, n, _ = xyz.shape
    xyz_t = jnp.transpose(xyz, (0, 2, 1))
    out = pl.pallas_call(
        functools.partial(_fps_body, npoint, b),
        grid=(1,),
        in_specs=[_rep_spec((b, n, 3)), _rep_spec((b, 3, n))],
        out_specs=_rep_spec((b, npoint, 1)),
        out_shape=jax.ShapeDtypeStruct((b, npoint, 1), jnp.int32),
    )(xyz, xyz_t)
    return out


def _knn_body(k, has_qidx, global_ofs, *refs):
    if has_qidx:
        xyzR_ref, xyzT_ref, qidx_ref, out_ref = refs
    else:
        xyzR_ref, xyzT_ref, out_ref = refs
    n = xyzT_ref.shape[-1]
    if has_qidx:
        nq = qidx_ref.shape[0]
        lane_q = jax.lax.broadcasted_iota(jnp.int32, (nq, n), 1)
        oh = (lane_q == qidx_ref[:, :]).astype(jnp.float32)
        q = jnp.dot(oh, xyzR_ref[...], preferred_element_type=jnp.float32)
    else:
        nq = xyzR_ref.shape[0]
        q = xyzR_ref[...]
    qx = q[:, 0:1]
    qy = q[:, 1:2]
    qz = q[:, 2:3]
    dx = qx - xyzT_ref[0:1, :]
    dy = qy - xyzT_ref[1:2, :]
    dz = qz - xyzT_ref[2:3, :]
    d = dx * dx + dy * dy + dz * dz
    lane = jax.lax.broadcasted_iota(jnp.int32, (nq, n), 1)
    big = jnp.float32(np.inf)
    ofs = pl.program_id(0) * n if global_ofs else 0
    for j in range(k):
        m = jnp.min(d, axis=1, keepdims=True)
        sel = jnp.where(d == m, lane, n)
        amin = jnp.min(sel, axis=1, keepdims=True)
        out_ref[:, pl.ds(j, 1)] = amin + ofs
        d = jnp.where(lane == amin, big, d)


def _knn_self(xyz, k, global_ofs=False):
    b, n, _ = xyz.shape
    xyz_t = jnp.transpose(xyz, (0, 2, 1))
    return pl.pallas_call(
        functools.partial(_knn_body, k, False, global_ofs),
        grid=(b,),
        in_specs=[_batch_spec((n, 3)), _batch_spec((3, n))],
        out_specs=_batch_spec((n, k)),
        out_shape=jax.ShapeDtypeStruct((b, n, k), jnp.int32),
    )(xyz, xyz_t)


def _knn_fps(xyz, qidx, k):
    b, n, _ = xyz.shape
    nq = qidx.shape[1]
    xyz_t = jnp.transpose(xyz, (0, 2, 1))
    return pl.pallas_call(
        functools.partial(_knn_body, k, True, False),
        grid=(b,),
        in_specs=[_batch_spec((n, 3)), _batch_spec((3, n)),
                  _batch_spec((nq, 1))],
        out_specs=_batch_spec((nq, k)),
        out_shape=jax.ShapeDtypeStruct((b, nq, k), jnp.int32),
    )(xyz, xyz_t, qidx)


def _tb_pre_body(xyzR_ref, f_ref, fc1w_ref, fc1b_ref, wq_ref, wk_ref,
                 wv_ref, d1w_ref, q_ref, t_ref):
    x = jnp.dot(f_ref[...], fc1w_ref[...],
                preferred_element_type=jnp.float32) + fc1b_ref[...]
    q_ref[...] = jnp.dot(x, wq_ref[...], preferred_element_type=jnp.float32)
    t_ref[:, 0:128] = jnp.dot(x, wk_ref[...],
                              preferred_element_type=jnp.float32)
    t_ref[:, 128:256] = jnp.dot(x, wv_ref[...],
                                preferred_element_type=jnp.float32)
    t_ref[:, 256:384] = jnp.dot(xyzR_ref[...], d1w_ref[...],
                                preferred_element_type=jnp.float32)


def _tb_post_body(k, tile, d1b_ref, d2w_ref, d2b_ref, g1w_ref, g1b_ref,
                  g2w_ref, g2b_ref, fc2w_ref, fc2b_ref, t_ref, q_ref,
                  knn_ref, pre_ref, out_ref, a_sc, w_sc):
    n = t_ref.shape[0]
    tid = pl.program_id(1)
    pq = t_ref[pl.ds(tid * tile, tile), 256:384]
    qv = q_ref[...]
    knn = knn_ref[...]
    lane = jax.lax.broadcasted_iota(jnp.int32, (tile, n), 1)
    table = t_ref[...]
    for j in range(k):
        idx = knn[:, j:j + 1]
        oh = (lane == idx).astype(jnp.float32)
        g = jnp.dot(oh, table, preferred_element_type=jnp.float32)
        xk = g[:, 0:128]
        xv = g[:, 128:256]
        pg = g[:, 256:384]
        pos = jnp.maximum(pq - pg + d1b_ref[...], 0.0)
        pos = jnp.dot(pos, d2w_ref[...],
                      preferred_element_type=jnp.float32) + d2b_ref[...]
        u = qv - xk + pos
        a = jnp.maximum(jnp.dot(u, g1w_ref[...],
                                preferred_element_type=jnp.float32)
                        + g1b_ref[...], 0.0)
        a = jnp.dot(a, g2w_ref[...],
                    preferred_element_type=jnp.float32) + g2b_ref[...]
        a_sc[j] = a / _SQRT_DM
        w_sc[j] = xv + pos
    m = a_sc[0]
    for j in range(1, k):
        m = jnp.maximum(m, a_sc[j])
    s = jnp.zeros((tile, _DM), jnp.float32)
    acc = jnp.zeros((tile, _DM), jnp.float32)
    for j in range(k):
        e = jnp.exp(a_sc[j] - m)
        s = s + e
        acc = acc + e * w_sc[j]
    res = acc / s
    out_ref[...] = (jnp.dot(res, fc2w_ref[...],
                            preferred_element_type=jnp.float32)
                    + fc2b_ref[...] + pre_ref[...])


def _tb_post_g_body(k, tile, d1b_ref, d2w_ref, d2b_ref, g1w_ref, g1b_ref,
                    g2w_ref, g2b_ref, fc2w_ref, fc2b_ref, g_ref, tq_ref,
                    q_ref, pre_ref, out_ref, a_sc, w_sc):
    pq = tq_ref[:, 256:384]
    qv = q_ref[...]
    for j in range(k):
        base = j * 384
        xk = g_ref[:, base:base + 128]
        xv = g_ref[:, base + 128:base + 256]
        pg = g_ref[:, base + 256:base + 384]
        pos = jnp.maximum(pq - pg + d1b_ref[...], 0.0)
        pos = jnp.dot(pos, d2w_ref[...],
                      preferred_element_type=jnp.float32) + d2b_ref[...]
        u = qv - xk + pos
        a = jnp.maximum(jnp.dot(u, g1w_ref[...],
                                preferred_element_type=jnp.float32)
                        + g1b_ref[...], 0.0)
        a = jnp.dot(a, g2w_ref[...],
                    preferred_element_type=jnp.float32) + g2b_ref[...]
        a_sc[j] = a / _SQRT_DM
        w_sc[j] = xv + pos
    m = a_sc[0]
    for j in range(1, k):
        m = jnp.maximum(m, a_sc[j])
    s = jnp.zeros((tile, _DM), jnp.float32)
    acc = jnp.zeros((tile, _DM), jnp.float32)
    for j in range(k):
        e = jnp.exp(a_sc[j] - m)
        s = s + e
        acc = acc + e * w_sc[j]
    res = acc / s
    out_ref[...] = (jnp.dot(res, fc2w_ref[...],
                            preferred_element_type=jnp.float32)
                    + fc2b_ref[...] + pre_ref[...])


def _tb(p, xyz, feats, knn, use_sc=False):
    b, n, d_in = feats.shape
    k = knn.shape[2]
    fc1w, fc1b = p['fc1']
    d1w, d1b = p['d1']
    d2w, d2b = p['d2']
    g1w, g1b = p['g1']
    g2w, g2b = p['g2']
    fc2w, fc2b = p['fc2']
    q, t = pl.pallas_call(
        _tb_pre_body,
        grid=(b,),
        in_specs=[_batch_spec((n, 3)), _batch_spec((n, d_in)),
                  _rep_spec(fc1w.shape), _rep_spec((1, _DM)),
                  _rep_spec(p['wq'].shape), _rep_spec(p['wk'].shape),
                  _rep_spec(p['wv'].shape), _rep_spec(d1w.shape)],
        out_specs=[_batch_spec((n, _DM)), _batch_spec((n, 384))],
        out_shape=[jax.ShapeDtypeStruct((b, n, _DM), jnp.float32),
                   jax.ShapeDtypeStruct((b, n, 384), jnp.float32)],
    )(xyz, feats, fc1w, fc1b.reshape(1, -1), p['wq'], p['wk'], p['wv'], d1w)

    if use_sc:
        chunk = _sc_chunk(b * n * k)
        g = _sc_gather(t.reshape(b * n, 384), knn.reshape(-1), chunk)
        gr = g.reshape(b, n, k * 384)
        tile = min(n, 128)
        nt = n // tile
        out = pl.pallas_call(
            functools.partial(_tb_post_g_body, k, tile),
            grid=(b, nt),
            in_specs=[_rep_spec((1, _DM)), _rep_spec(d2w.shape),
                      _rep_spec((1, _DM)), _rep_spec(g1w.shape),
                      _rep_spec((1, _DM)), _rep_spec(g2w.shape),
                      _rep_spec((1, _DM)), _rep_spec(fc2w.shape),
                      _rep_spec((1, d_in)),
                      pl.BlockSpec((None, tile, k * 384),
                                   lambda b_, t_: (b_, t_, 0)),
                      pl.BlockSpec((None, tile, 384),
                                   lambda b_, t_: (b_, t_, 0)),
                      pl.BlockSpec((None, tile, _DM),
                                   lambda b_, t_: (b_, t_, 0)),
                      pl.BlockSpec((None, tile, d_in),
                                   lambda b_, t_: (b_, t_, 0))],
            out_specs=pl.BlockSpec((None, tile, d_in),
                                   lambda b_, t_: (b_, t_, 0)),
            out_shape=jax.ShapeDtypeStruct((b, n, d_in), jnp.float32),
            scratch_shapes=[pltpu.VMEM((k, tile, _DM), jnp.float32),
                            pltpu.VMEM((k, tile, _DM), jnp.float32)],
        )(d1b.reshape(1, -1), d2w, d2b.reshape(1, -1), g1w,
          g1b.reshape(1, -1), g2w, g2b.reshape(1, -1), fc2w,
          fc2b.reshape(1, -1), gr, t, q, feats)
        return out

    tile = min(n, 256)
    nt = n // tile
    out = pl.pallas_call(
        functools.partial(_tb_post_body, k, tile),
        grid=(b, nt),
        in_specs=[_rep_spec((1, _DM)), _rep_spec(d2w.shape),
                  _rep_spec((1, _DM)), _rep_spec(g1w.shape),
                  _rep_spec((1, _DM)), _rep_spec(g2w.shape),
                  _rep_spec((1, _DM)), _rep_spec(fc2w.shape),
                  _rep_spec((1, d_in)),
                  pl.BlockSpec((None, n, 384), lambda b_, t_: (b_, 0, 0)),
                  pl.BlockSpec((None, tile, _DM), lambda b_, t_: (b_, t_, 0)),
                  pl.BlockSpec((None, tile, k), lambda b_, t_: (b_, t_, 0)),
                  pl.BlockSpec((None, tile, d_in), lambda b_, t_: (b_, t_, 0))],
        out_specs=pl.BlockSpec((None, tile, d_in), lambda b_, t_: (b_, t_, 0)),
        out_shape=jax.ShapeDtypeStruct((b, n, d_in), jnp.float32),
        scratch_shapes=[pltpu.VMEM((k, tile, _DM), jnp.float32),
                        pltpu.VMEM((k, tile, _DM), jnp.float32)],
    )(d1b.reshape(1, -1), d2w, d2b.reshape(1, -1), g1w, g1b.reshape(1, -1),
      g2w, g2b.reshape(1, -1), fc2w, fc2b.reshape(1, -1), t, q, knn, feats)
    return out


def _td_body(k, xyzR_ref, f_ref, fps_ref, knn_ref, l1wx_ref, l1wf_ref,
             l1b_ref, l2w_ref, l2b_ref, nxyz_ref, out_ref):
    n = xyzR_ref.shape[0]
    npt = fps_ref.shape[0]
    c_out = l2w_ref.shape[0]
    lane = jax.lax.broadcasted_iota(jnp.int32, (npt, n), 1)
    oh_fps = (lane == fps_ref[:, :]).astype(jnp.float32)
    new_xyz = jnp.dot(oh_fps, xyzR_ref[...],
                      preferred_element_type=jnp.float32)
    nxyz_ref[...] = new_xyz
    knn = knn_ref[...]
    m = jnp.full((npt, c_out), -jnp.inf, jnp.float32)
    for j in range(k):
        idx = knn[:, j:j + 1]
        oh = (lane == idx).astype(jnp.float32)
        gx = jnp.dot(oh, xyzR_ref[...],
                     preferred_element_type=jnp.float32) - new_xyz
        gf = jnp.dot(oh, f_ref[...], preferred_element_type=jnp.float32)
        h = (jnp.dot(gx, l1wx_ref[...], preferred_element_type=jnp.float32)
             + jnp.dot(gf, l1wf_ref[...], preferred_element_type=jnp.float32)
             + l1b_ref[...])
        h = jnp.maximum(h, 0.0)
        h = jnp.dot(h, l2w_ref[...],
                    preferred_element_type=jnp.float32) + l2b_ref[...]
        h = jnp.maximum(h, 0.0)
        m = jnp.maximum(m, h)
    out_ref[...] = m


def _td(p, xyz, feats, fps, knn):
    b, n, c_in = feats.shape
    npt = fps.shape[1]
    k = knn.shape[2]
    l1w, l1b = p['l1']
    l2w, l2b = p['l2']
    c_out = l2w.shape[1]
    nxyz, f_out = pl.pallas_call(
        functools.partial(_td_body, k),
        grid=(b,),
        in_specs=[_batch_spec((n, 3)), _batch_spec((n, c_in)),
                  _batch_spec((npt, 1)), _batch_spec((npt, k)),
                  _rep_spec((3, c_out)), _rep_spec((c_in, c_out)),
                  _rep_spec((1, c_out)), _rep_spec(l2w.shape),
                  _rep_spec((1, c_out))],
        out_specs=[_batch_spec((npt, 3)), _batch_spec((npt, c_out))],
        out_shape=[jax.ShapeDtypeStruct((b, npt, 3), jnp.float32),
                   jax.ShapeDtypeStruct((b, npt, c_out), jnp.float32)],
    )(xyz, feats, fps, knn, l1w[:3], l1w[3:], l1b.reshape(1, -1),
      l2w, l2b.reshape(1, -1))
    return nxyz, f_out


def _tu_body(fc_ref, xycR_ref, xycT_ref, ff_ref, xyf_ref, w1_ref, b1_ref,
             w2_ref, b2_ref, out_ref):
    nc = xycR_ref.shape[0]
    nf = xyf_ref.shape[0]
    f1 = jnp.maximum(jnp.dot(fc_ref[...], w1_ref[...],
                             preferred_element_type=jnp.float32)
                     + b1_ref[...], 0.0)
    f2 = jnp.maximum(jnp.dot(ff_ref[...], w2_ref[...],
                             preferred_element_type=jnp.float32)
                     + b2_ref[...], 0.0)
    dx = xyf_ref[:, 0:1] - xycT_ref[0:1, :]
    dy = xyf_ref[:, 1:2] - xycT_ref[1:2, :]
    dz = xyf_ref[:, 2:3] - xycT_ref[2:3, :]
    d = dx * dx + dy * dy + dz * dz
    lane = jax.lax.broadcasted_iota(jnp.int32, (nf, nc), 1)
    big = jnp.float32(np.inf)
    ws = []
    idxs = []
    for j in range(3):
        m = jnp.min(d, axis=1, keepdims=True)
        sel = jnp.where(d == m, lane, nc)
        amin = jnp.min(sel, axis=1, keepdims=True)
        ws.append(1.0 / jnp.maximum(m, 1e-10))
        idxs.append(amin)
        d = jnp.where(lane == amin, big, d)
    wsum = (ws[0] + ws[1]) + ws[2]
    acc = None
    for j in range(3):
        oh = (lane == idxs[j]).astype(jnp.float32)
        fj = jnp.dot(oh, f1, preferred_element_type=jnp.float32)
        term = (ws[j] / wsum) * fj
        acc = term if acc is None else acc + term
    out_ref[...] = acc + f2


def _tu(p, f_coarse, xyz_coarse, f_fine, xyz_fine):
    b, nc, _ = xyz_coarse.shape
    nf = xyz_fine.shape[1]
    w1, b1 = p['fc1']
    w2, b2 = p['fc2']
    d = w1.shape[1]
    xyc_t = jnp.transpose(xyz_coarse, (0, 2, 1))
    return pl.pallas_call(
        _tu_body,
        grid=(b,),
        in_specs=[_batch_spec(f_coarse.shape[1:]), _batch_spec((nc, 3)),
                  _batch_spec((3, nc)), _batch_spec(f_fine.shape[1:]),
                  _batch_spec((nf, 3)), _rep_spec(w1.shape),
                  _rep_spec((1, d)), _rep_spec(w2.shape), _rep_spec((1, d))],
        out_specs=_batch_spec((nf, d)),
        out_shape=jax.ShapeDtypeStruct((b, nf, d), jnp.float32),
    )(f_coarse, xyz_coarse, xyc_t, f_fine, xyz_fine, w1,
      b1.reshape(1, -1), w2, b2.reshape(1, -1))


def _mlp_body(relus, nlayer, *refs):
    x_ref = refs[0]
    out_ref = refs[-1]
    h = x_ref[...]
    for i in range(nlayer):
        w_ref = refs[1 + 2 * i]
        b_ref = refs[2 + 2 * i]
        h = jnp.dot(h, w_ref[...],
                    preferred_element_type=jnp.float32) + b_ref[...]
        if relus[i]:
            h = jnp.maximum(h, 0.0)
    out_ref[...] = h


def _mlp(x, layers, relus):
    b, n, _ = x.shape
    nlayer = len(layers)
    args = [x]
    specs = [_batch_spec(x.shape[1:])]
    for (w, bias) in layers:
        args.append(w)
        args.append(bias.reshape(1, -1))
        specs.append(_rep_spec(w.shape))
        specs.append(_rep_spec((1, w.shape[1])))
    d_out = layers[-1][0].shape[1]
    return pl.pallas_call(
        functools.partial(_mlp_body, relus, nlayer),
        grid=(b,),
        in_specs=specs,
        out_specs=_batch_spec((n, d_out)),
        out_shape=jax.ShapeDtypeStruct((b, n, d_out), jnp.float32),
    )(*args)


def _tb_stage(p, xyz, feats):
    n = xyz.shape[1]
    k = min(_KP, n)
    use_sc = False
    knn = _knn_self(xyz, k, global_ofs=use_sc)
    return _tb(p, xyz, feats, knn, use_sc=use_sc)


def kernel(x, params):
    xb = jnp.transpose(x, (0, 2, 1))
    xyz = xb
    f = _mlp(xb, [params['bb_fc1a'], params['bb_fc1b']], [True, False])
    f = _tb_stage(params['bb_tb0'], xyz, f)
    fac = [(f, xyz)]
    npts = xyz.shape[1]
    for i in range(4):
        npts //= 4
        fps = _fps(xyz, npts)
        knn_d = _knn_fps(xyz, fps, _KP)
        xyz, f = _td(params['bb_td'][i], xyz, f, fps, knn_d)
        f = _tb_stage(params['bb_tbs'][i], xyz, f)
        fac.append((f, xyz))
    feature, coord = fac[-1]
    h = _mlp(feature, [params['mlp2a'], params['mlp2b'], params['mlp2c']],
             [True, True, False])
    feature = _tb_stage(params['t2'], coord, h)
    for i in range(4):
        f_fine, c_fine = fac[-i - 2]
        feature = _tu(params['tu'][i], feature, coord, f_fine, c_fine)
        coord = c_fine
        feature = _tb_stage(params['tbu'][i], coord, feature)
    h = _mlp(feature, [params['mlp3a'], params['mlp3b'], params['mlp3c']],
             [True, True, False])
    return h


# speedup vs baseline: 1.7773x; 1.1545x over previous
import functools

import jax
import jax.numpy as jnp
import numpy as np
from jax.experimental import pallas as pl
from jax.experimental.pallas import tpu as pltpu
from jax.experimental.pallas import tpu_sc as plsc

_B = 2
_KP = 16
_DM = 128
_SQRT_DM = np.float32(np.sqrt(128.0))

_SC_NC = 2
_SC_NS = 16
_SC_NW = _SC_NC * _SC_NS


def _sc_gather(table, idx, chunk):
    bn = idx.shape[0]
    d = table.shape[1]
    per_w = bn // (chunk * _SC_NW)
    mesh = plsc.VectorSubcoreMesh(core_axis_name="c", subcore_axis_name="s",
                                  num_cores=_SC_NC, num_subcores=_SC_NS)

    def body(table_hbm, idx_hbm, out_hbm, idx_v, rows_v, sem):
        wid = jax.lax.axis_index("s") * _SC_NC + jax.lax.axis_index("c")
        for j in range(per_w):
            base = (wid * per_w + j) * chunk
            pltpu.sync_copy(idx_hbm.at[pl.ds(base, chunk)], idx_v)
            pltpu.async_copy(table_hbm.at[idx_v], rows_v, sem).wait()
            pltpu.sync_copy(rows_v, out_hbm.at[pl.ds(base, chunk)])

    f = pl.kernel(
        body,
        out_type=jax.ShapeDtypeStruct((bn, d), jnp.float32),
        mesh=mesh,
        scratch_types=[pltpu.VMEM((chunk,), jnp.int32),
                       pltpu.VMEM((chunk, d), jnp.float32),
                       pltpu.SemaphoreType.DMA],
    )
    return f(table, idx)


def _sc_chunk(bn):
    for c in (128, 64, 32, 16, 8):
        if bn % (c * _SC_NW) == 0:
            return c
    return 0


def _rep_spec(shape):
    nd = len(shape)
    return pl.BlockSpec(shape, lambda *_: (0,) * nd)


def _batch_spec(shape):
    nd = len(shape)
    return pl.BlockSpec((None,) + shape, lambda b: (b,) + (0,) * nd)


def _fps_body(npoint, bsz, xyzR_ref, xyzT_ref, out_ref):
    n = xyzT_ref.shape[-1]
    lane = jax.lax.broadcasted_iota(jnp.int32, (1, n), 1)
    xs = [xyzT_ref[b, 0:1, :] for b in range(bsz)]
    ys = [xyzT_ref[b, 1:2, :] for b in range(bsz)]
    zs = [xyzT_ref[b, 2:3, :] for b in range(bsz)]

    ninf = jnp.float32(-np.inf)

    def body(i, carry):
        dists, fars = carry
        new_d, new_f = [], []
        for b in range(bsz):
            out_ref[b, pl.ds(i, 1), :] = fars[b]
            sel_mask = lane == fars[b]
            cx = jnp.max(jnp.where(sel_mask, xs[b], ninf),
                         axis=1, keepdims=True)
            cy = jnp.max(jnp.where(sel_mask, ys[b], ninf),
                         axis=1, keepdims=True)
            cz = jnp.max(jnp.where(sel_mask, zs[b], ninf),
                         axis=1, keepdims=True)
            dx = xs[b] - cx
            dy = ys[b] - cy
            dz = zs[b] - cz
            dist = dx * dx + dy * dy + dz * dz
            dist_min = jnp.minimum(dists[b], dist)
            m = jnp.max(dist_min, axis=1, keepdims=True)
            sel = jnp.where(dist_min == m, lane, n)
            new_d.append(dist_min)
            new_f.append(jnp.min(sel, axis=1, keepdims=True))
        return tuple(new_d), tuple(new_f)

    init = (tuple(jnp.full((1, n), 1e10, dtype=jnp.float32)
                  for _ in range(bsz)),
            tuple(jnp.zeros((1, 1), dtype=jnp.int32) for _ in range(bsz)))
    jax.lax.fori_loop(0, npoint, body, init)


def _fps(xyz, npoint):
    b, n, _ = xyz.shape
    xyz_t = jnp.transpose(xyz, (0, 2, 1))
    out = pl.pallas_call(
        functools.partial(_fps_body, npoint, b),
        grid=(1,),
        in_specs=[_rep_spec((b, n, 3)), _rep_spec((b, 3, n))],
        out_specs=_rep_spec((b, npoint, 1)),
        out_shape=jax.ShapeDtypeStruct((b, npoint, 1), jnp.int32),
    )(xyz, xyz_t)
    return out


def _knn_body(k, has_qidx, global_ofs, *refs):
    if has_qidx:
        xyzR_ref, xyzT_ref, qidx_ref, out_ref = refs
    else:
        xyzR_ref, xyzT_ref, out_ref = refs
    n = xyzT_ref.shape[-1]
    if has_qidx:
        nq = qidx_ref.shape[0]
        lane_q = jax.lax.broadcasted_iota(jnp.int32, (nq, n), 1)
        oh = (lane_q == qidx_ref[:, :]).astype(jnp.float32)
        q = jnp.dot(oh, xyzR_ref[...], preferred_element_type=jnp.float32)
    else:
        nq = xyzR_ref.shape[0]
        q = xyzR_ref[...]
    qx = q[:, 0:1]
    qy = q[:, 1:2]
    qz = q[:, 2:3]
    dx = qx - xyzT_ref[0:1, :]
    dy = qy - xyzT_ref[1:2, :]
    dz = qz - xyzT_ref[2:3, :]
    d = dx * dx + dy * dy + dz * dz
    lane = jax.lax.broadcasted_iota(jnp.int32, (nq, n), 1)
    big = jnp.float32(np.inf)
    ofs = pl.program_id(0) * n if global_ofs else 0
    for j in range(k):
        m = jnp.min(d, axis=1, keepdims=True)
        sel = jnp.where(d == m, lane, n)
        amin = jnp.min(sel, axis=1, keepdims=True)
        out_ref[:, pl.ds(j, 1)] = amin + ofs
        d = jnp.where(lane == amin, big, d)


def _knn_self(xyz, k, global_ofs=False):
    b, n, _ = xyz.shape
    xyz_t = jnp.transpose(xyz, (0, 2, 1))
    return pl.pallas_call(
        functools.partial(_knn_body, k, False, global_ofs),
        grid=(b,),
        in_specs=[_batch_spec((n, 3)), _batch_spec((3, n))],
        out_specs=_batch_spec((n, k)),
        out_shape=jax.ShapeDtypeStruct((b, n, k), jnp.int32),
    )(xyz, xyz_t)


def _knn_fps(xyz, qidx, k):
    b, n, _ = xyz.shape
    nq = qidx.shape[1]
    xyz_t = jnp.transpose(xyz, (0, 2, 1))
    return pl.pallas_call(
        functools.partial(_knn_body, k, True, False),
        grid=(b,),
        in_specs=[_batch_spec((n, 3)), _batch_spec((3, n)),
                  _batch_spec((nq, 1))],
        out_specs=_batch_spec((nq, k)),
        out_shape=jax.ShapeDtypeStruct((b, nq, k), jnp.int32),
    )(xyz, xyz_t, qidx)


def _tb_pre_body(xyzR_ref, f_ref, fc1w_ref, fc1b_ref, wq_ref, wk_ref,
                 wv_ref, d1w_ref, q_ref, t_ref):
    x = jnp.dot(f_ref[...], fc1w_ref[...],
                preferred_element_type=jnp.float32) + fc1b_ref[...]
    q_ref[...] = jnp.dot(x, wq_ref[...], preferred_element_type=jnp.float32)
    t_ref[:, 0:128] = jnp.dot(x, wk_ref[...],
                              preferred_element_type=jnp.float32)
    t_ref[:, 128:256] = jnp.dot(x, wv_ref[...],
                                preferred_element_type=jnp.float32)
    t_ref[:, 256:384] = jnp.dot(xyzR_ref[...], d1w_ref[...],
                                preferred_element_type=jnp.float32)


def _tb_post_body(k, tile, d1b_ref, d2w_ref, d2b_ref, g1w_ref, g1b_ref,
                  g2w_ref, g2b_ref, fc2w_ref, fc2b_ref, t_ref, q_ref,
                  knn_ref, pre_ref, out_ref):
    n = t_ref.shape[0]
    kt = k * tile
    tid = pl.program_id(1)
    pq = t_ref[pl.ds(tid * tile, tile), 256:384]
    qv = q_ref[...]
    knn = knn_ref[...]
    idx_s = jnp.concatenate([knn[:, j:j + 1] for j in range(k)], axis=0)
    lane = jax.lax.broadcasted_iota(jnp.int32, (kt, n), 1)
    oh = (lane == idx_s).astype(jnp.float32)
    g = jnp.dot(oh, t_ref[...], preferred_element_type=jnp.float32)
    pq_s = jnp.concatenate([pq] * k, axis=0)
    q_s = jnp.concatenate([qv] * k, axis=0)
    pos = jnp.maximum(pq_s - g[:, 256:384] + d1b_ref[...], 0.0)
    pos = jnp.dot(pos, d2w_ref[...],
                  preferred_element_type=jnp.float32) + d2b_ref[...]
    u = q_s - g[:, 0:128] + pos
    a = jnp.maximum(jnp.dot(u, g1w_ref[...],
                            preferred_element_type=jnp.float32)
                    + g1b_ref[...], 0.0)
    a = (jnp.dot(a, g2w_ref[...],
                 preferred_element_type=jnp.float32) + g2b_ref[...])
    a = a / _SQRT_DM
    w = g[:, 128:256] + pos
    m = a[0:tile, :]
    for j in range(1, k):
        m = jnp.maximum(m, a[j * tile:(j + 1) * tile, :])
    s = jnp.zeros((tile, _DM), jnp.float32)
    acc = jnp.zeros((tile, _DM), jnp.float32)
    for j in range(k):
        e = jnp.exp(a[j * tile:(j + 1) * tile, :] - m)
        s = s + e
        acc = acc + e * w[j * tile:(j + 1) * tile, :]
    res = acc / s
    out_ref[...] = (jnp.dot(res, fc2w_ref[...],
                            preferred_element_type=jnp.float32)
                    + fc2b_ref[...] + pre_ref[...])


def _tb_post_g_body(k, tile, d1b_ref, d2w_ref, d2b_ref, g1w_ref, g1b_ref,
                    g2w_ref, g2b_ref, fc2w_ref, fc2b_ref, g_ref, tq_ref,
                    q_ref, pre_ref, out_ref, a_sc, w_sc):
    pq = tq_ref[:, 256:384]
    qv = q_ref[...]
    for j in range(k):
        base = j * 384
        xk = g_ref[:, base:base + 128]
        xv = g_ref[:, base + 128:base + 256]
        pg = g_ref[:, base + 256:base + 384]
        pos = jnp.maximum(pq - pg + d1b_ref[...], 0.0)
        pos = jnp.dot(pos, d2w_ref[...],
                      preferred_element_type=jnp.float32) + d2b_ref[...]
        u = qv - xk + pos
        a = jnp.maximum(jnp.dot(u, g1w_ref[...],
                                preferred_element_type=jnp.float32)
                        + g1b_ref[...], 0.0)
        a = jnp.dot(a, g2w_ref[...],
                    preferred_element_type=jnp.float32) + g2b_ref[...]
        a_sc[j] = a / _SQRT_DM
        w_sc[j] = xv + pos
    m = a_sc[0]
    for j in range(1, k):
        m = jnp.maximum(m, a_sc[j])
    s = jnp.zeros((tile, _DM), jnp.float32)
    acc = jnp.zeros((tile, _DM), jnp.float32)
    for j in range(k):
        e = jnp.exp(a_sc[j] - m)
        s = s + e
        acc = acc + e * w_sc[j]
    res = acc / s
    out_ref[...] = (jnp.dot(res, fc2w_ref[...],
                            preferred_element_type=jnp.float32)
                    + fc2b_ref[...] + pre_ref[...])


def _tb(p, xyz, feats, knn, use_sc=False):
    b, n, d_in = feats.shape
    k = knn.shape[2]
    fc1w, fc1b = p['fc1']
    d1w, d1b = p['d1']
    d2w, d2b = p['d2']
    g1w, g1b = p['g1']
    g2w, g2b = p['g2']
    fc2w, fc2b = p['fc2']
    q, t = pl.pallas_call(
        _tb_pre_body,
        grid=(b,),
        in_specs=[_batch_spec((n, 3)), _batch_spec((n, d_in)),
                  _rep_spec(fc1w.shape), _rep_spec((1, _DM)),
                  _rep_spec(p['wq'].shape), _rep_spec(p['wk'].shape),
                  _rep_spec(p['wv'].shape), _rep_spec(d1w.shape)],
        out_specs=[_batch_spec((n, _DM)), _batch_spec((n, 384))],
        out_shape=[jax.ShapeDtypeStruct((b, n, _DM), jnp.float32),
                   jax.ShapeDtypeStruct((b, n, 384), jnp.float32)],
    )(xyz, feats, fc1w, fc1b.reshape(1, -1), p['wq'], p['wk'], p['wv'], d1w)

    if use_sc:
        chunk = _sc_chunk(b * n * k)
        g = _sc_gather(t.reshape(b * n, 384), knn.reshape(-1), chunk)
        gr = g.reshape(b, n, k * 384)
        tile = min(n, 128)
        nt = n // tile
        out = pl.pallas_call(
            functools.partial(_tb_post_g_body, k, tile),
            grid=(b, nt),
            in_specs=[_rep_spec((1, _DM)), _rep_spec(d2w.shape),
                      _rep_spec((1, _DM)), _rep_spec(g1w.shape),
                      _rep_spec((1, _DM)), _rep_spec(g2w.shape),
                      _rep_spec((1, _DM)), _rep_spec(fc2w.shape),
                      _rep_spec((1, d_in)),
                      pl.BlockSpec((None, tile, k * 384),
                                   lambda b_, t_: (b_, t_, 0)),
                      pl.BlockSpec((None, tile, 384),
                                   lambda b_, t_: (b_, t_, 0)),
                      pl.BlockSpec((None, tile, _DM),
                                   lambda b_, t_: (b_, t_, 0)),
                      pl.BlockSpec((None, tile, d_in),
                                   lambda b_, t_: (b_, t_, 0))],
            out_specs=pl.BlockSpec((None, tile, d_in),
                                   lambda b_, t_: (b_, t_, 0)),
            out_shape=jax.ShapeDtypeStruct((b, n, d_in), jnp.float32),
            scratch_shapes=[pltpu.VMEM((k, tile, _DM), jnp.float32),
                            pltpu.VMEM((k, tile, _DM), jnp.float32)],
        )(d1b.reshape(1, -1), d2w, d2b.reshape(1, -1), g1w,
          g1b.reshape(1, -1), g2w, g2b.reshape(1, -1), fc2w,
          fc2b.reshape(1, -1), gr, t, q, feats)
        return out

    tile = min(n, max(1024 // k, 8))
    nt = n // tile
    out = pl.pallas_call(
        functools.partial(_tb_post_body, k, tile),
        grid=(b, nt),
        in_specs=[_rep_spec((1, _DM)), _rep_spec(d2w.shape),
                  _rep_spec((1, _DM)), _rep_spec(g1w.shape),
                  _rep_spec((1, _DM)), _rep_spec(g2w.shape),
                  _rep_spec((1, _DM)), _rep_spec(fc2w.shape),
                  _rep_spec((1, d_in)),
                  pl.BlockSpec((None, n, 384), lambda b_, t_: (b_, 0, 0)),
                  pl.BlockSpec((None, tile, _DM), lambda b_, t_: (b_, t_, 0)),
                  pl.BlockSpec((None, tile, k), lambda b_, t_: (b_, t_, 0)),
                  pl.BlockSpec((None, tile, d_in), lambda b_, t_: (b_, t_, 0))],
        out_specs=pl.BlockSpec((None, tile, d_in), lambda b_, t_: (b_, t_, 0)),
        out_shape=jax.ShapeDtypeStruct((b, n, d_in), jnp.float32),
    )(d1b.reshape(1, -1), d2w, d2b.reshape(1, -1), g1w, g1b.reshape(1, -1),
      g2w, g2b.reshape(1, -1), fc2w, fc2b.reshape(1, -1), t, q, knn, feats)
    return out


def _td_body(k, xyzR_ref, f_ref, fps_ref, knn_ref, l1wx_ref, l1wf_ref,
             l1b_ref, l2w_ref, l2b_ref, nxyz_ref, out_ref):
    n = xyzR_ref.shape[0]
    npt = fps_ref.shape[0]
    c_out = l2w_ref.shape[0]
    lane = jax.lax.broadcasted_iota(jnp.int32, (npt, n), 1)
    oh_fps = (lane == fps_ref[:, :]).astype(jnp.float32)
    new_xyz = jnp.dot(oh_fps, xyzR_ref[...],
                      preferred_element_type=jnp.float32)
    nxyz_ref[...] = new_xyz
    knn = knn_ref[...]
    m = jnp.full((npt, c_out), -jnp.inf, jnp.float32)
    for j in range(k):
        idx = knn[:, j:j + 1]
        oh = (lane == idx).astype(jnp.float32)
        gx = jnp.dot(oh, xyzR_ref[...],
                     preferred_element_type=jnp.float32) - new_xyz
        gf = jnp.dot(oh, f_ref[...], preferred_element_type=jnp.float32)
        h = (jnp.dot(gx, l1wx_ref[...], preferred_element_type=jnp.float32)
             + jnp.dot(gf, l1wf_ref[...], preferred_element_type=jnp.float32)
             + l1b_ref[...])
        h = jnp.maximum(h, 0.0)
        h = jnp.dot(h, l2w_ref[...],
                    preferred_element_type=jnp.float32) + l2b_ref[...]
        h = jnp.maximum(h, 0.0)
        m = jnp.maximum(m, h)
    out_ref[...] = m


def _td(p, xyz, feats, fps, knn):
    b, n, c_in = feats.shape
    npt = fps.shape[1]
    k = knn.shape[2]
    l1w, l1b = p['l1']
    l2w, l2b = p['l2']
    c_out = l2w.shape[1]
    nxyz, f_out = pl.pallas_call(
        functools.partial(_td_body, k),
        grid=(b,),
        in_specs=[_batch_spec((n, 3)), _batch_spec((n, c_in)),
                  _batch_spec((npt, 1)), _batch_spec((npt, k)),
                  _rep_spec((3, c_out)), _rep_spec((c_in, c_out)),
                  _rep_spec((1, c_out)), _rep_spec(l2w.shape),
                  _rep_spec((1, c_out))],
        out_specs=[_batch_spec((npt, 3)), _batch_spec((npt, c_out))],
        out_shape=[jax.ShapeDtypeStruct((b, npt, 3), jnp.float32),
                   jax.ShapeDtypeStruct((b, npt, c_out), jnp.float32)],
    )(xyz, feats, fps, knn, l1w[:3], l1w[3:], l1b.reshape(1, -1),
      l2w, l2b.reshape(1, -1))
    return nxyz, f_out


def _tu_body(fc_ref, xycR_ref, xycT_ref, ff_ref, xyf_ref, w1_ref, b1_ref,
             w2_ref, b2_ref, out_ref):
    nc = xycR_ref.shape[0]
    nf = xyf_ref.shape[0]
    f1 = jnp.maximum(jnp.dot(fc_ref[...], w1_ref[...],
                             preferred_element_type=jnp.float32)
                     + b1_ref[...], 0.0)
    f2 = jnp.maximum(jnp.dot(ff_ref[...], w2_ref[...],
                             preferred_element_type=jnp.float32)
                     + b2_ref[...], 0.0)
    dx = xyf_ref[:, 0:1] - xycT_ref[0:1, :]
    dy = xyf_ref[:, 1:2] - xycT_ref[1:2, :]
    dz = xyf_ref[:, 2:3] - xycT_ref[2:3, :]
    d = dx * dx + dy * dy + dz * dz
    lane = jax.lax.broadcasted_iota(jnp.int32, (nf, nc), 1)
    big = jnp.float32(np.inf)
    ws = []
    idxs = []
    for j in range(3):
        m = jnp.min(d, axis=1, keepdims=True)
        sel = jnp.where(d == m, lane, nc)
        amin = jnp.min(sel, axis=1, keepdims=True)
        ws.append(1.0 / jnp.maximum(m, 1e-10))
        idxs.append(amin)
        d = jnp.where(lane == amin, big, d)
    wsum = (ws[0] + ws[1]) + ws[2]
    acc = None
    for j in range(3):
        oh = (lane == idxs[j]).astype(jnp.float32)
        fj = jnp.dot(oh, f1, preferred_element_type=jnp.float32)
        term = (ws[j] / wsum) * fj
        acc = term if acc is None else acc + term
    out_ref[...] = acc + f2


def _tu(p, f_coarse, xyz_coarse, f_fine, xyz_fine):
    b, nc, _ = xyz_coarse.shape
    nf = xyz_fine.shape[1]
    w1, b1 = p['fc1']
    w2, b2 = p['fc2']
    d = w1.shape[1]
    xyc_t = jnp.transpose(xyz_coarse, (0, 2, 1))
    return pl.pallas_call(
        _tu_body,
        grid=(b,),
        in_specs=[_batch_spec(f_coarse.shape[1:]), _batch_spec((nc, 3)),
                  _batch_spec((3, nc)), _batch_spec(f_fine.shape[1:]),
                  _batch_spec((nf, 3)), _rep_spec(w1.shape),
                  _rep_spec((1, d)), _rep_spec(w2.shape), _rep_spec((1, d))],
        out_specs=_batch_spec((nf, d)),
        out_shape=jax.ShapeDtypeStruct((b, nf, d), jnp.float32),
    )(f_coarse, xyz_coarse, xyc_t, f_fine, xyz_fine, w1,
      b1.reshape(1, -1), w2, b2.reshape(1, -1))


def _mlp_body(relus, nlayer, *refs):
    x_ref = refs[0]
    out_ref = refs[-1]
    h = x_ref[...]
    for i in range(nlayer):
        w_ref = refs[1 + 2 * i]
        b_ref = refs[2 + 2 * i]
        h = jnp.dot(h, w_ref[...],
                    preferred_element_type=jnp.float32) + b_ref[...]
        if relus[i]:
            h = jnp.maximum(h, 0.0)
    out_ref[...] = h


def _mlp(x, layers, relus):
    b, n, _ = x.shape
    nlayer = len(layers)
    args = [x]
    specs = [_batch_spec(x.shape[1:])]
    for (w, bias) in layers:
        args.append(w)
        args.append(bias.reshape(1, -1))
        specs.append(_rep_spec(w.shape))
        specs.append(_rep_spec((1, w.shape[1])))
    d_out = layers[-1][0].shape[1]
    return pl.pallas_call(
        functools.partial(_mlp_body, relus, nlayer),
        grid=(b,),
        in_specs=specs,
        out_specs=_batch_spec((n, d_out)),
        out_shape=jax.ShapeDtypeStruct((b, n, d_out), jnp.float32),
    )(*args)


def _tb_stage(p, xyz, feats):
    n = xyz.shape[1]
    k = min(_KP, n)
    use_sc = False
    knn = _knn_self(xyz, k, global_ofs=use_sc)
    return _tb(p, xyz, feats, knn, use_sc=use_sc)


def kernel(x, params):
    xb = jnp.transpose(x, (0, 2, 1))
    xyz = xb
    f = _mlp(xb, [params['bb_fc1a'], params['bb_fc1b']], [True, False])
    f = _tb_stage(params['bb_tb0'], xyz, f)
    fac = [(f, xyz)]
    npts = xyz.shape[1]
    for i in range(4):
        npts //= 4
        fps = _fps(xyz, npts)
        knn_d = _knn_fps(xyz, fps, _KP)
        xyz, f = _td(params['bb_td'][i], xyz, f, fps, knn_d)
        f = _tb_stage(params['bb_tbs'][i], xyz, f)
        fac.append((f, xyz))
    feature, coord = fac[-1]
    h = _mlp(feature, [params['mlp2a'], params['mlp2b'], params['mlp2c']],
             [True, True, False])
    feature = _tb_stage(params['t2'], coord, h)
    for i in range(4):
        f_fine, c_fine = fac[-i - 2]
        feature = _tu(params['tu'][i], feature, coord, f_fine, c_fine)
        coord = c_fine
        feature = _tb_stage(params['tbu'][i], coord, feature)
    h = _mlp(feature, [params['mlp3a'], params['mlp3b'], params['mlp3c']],
             [True, True, False])
    return h


# speedup vs baseline: 1.8519x; 1.0420x over previous
import functools

import jax
import jax.numpy as jnp
import numpy as np
from jax.experimental import pallas as pl
from jax.experimental.pallas import tpu as pltpu
from jax.experimental.pallas import tpu_sc as plsc

_B = 2
_KP = 16
_DM = 128
_SQRT_DM = np.float32(np.sqrt(128.0))

_SC_NC = 2
_SC_NS = 16
_SC_NW = _SC_NC * _SC_NS


def _sc_gather(table, idx, chunk):
    bn = idx.shape[0]
    d = table.shape[1]
    per_w = bn // (chunk * _SC_NW)
    mesh = plsc.VectorSubcoreMesh(core_axis_name="c", subcore_axis_name="s",
                                  num_cores=_SC_NC, num_subcores=_SC_NS)

    def body(table_hbm, idx_hbm, out_hbm, idx_v, rows_v, sem):
        wid = jax.lax.axis_index("s") * _SC_NC + jax.lax.axis_index("c")
        for j in range(per_w):
            base = (wid * per_w + j) * chunk
            pltpu.sync_copy(idx_hbm.at[pl.ds(base, chunk)], idx_v)
            pltpu.async_copy(table_hbm.at[idx_v], rows_v, sem).wait()
            pltpu.sync_copy(rows_v, out_hbm.at[pl.ds(base, chunk)])

    f = pl.kernel(
        body,
        out_type=jax.ShapeDtypeStruct((bn, d), jnp.float32),
        mesh=mesh,
        scratch_types=[pltpu.VMEM((chunk,), jnp.int32),
                       pltpu.VMEM((chunk, d), jnp.float32),
                       pltpu.SemaphoreType.DMA],
    )
    return f(table, idx)


def _sc_chunk(bn):
    for c in (128, 64, 32, 16, 8):
        if bn % (c * _SC_NW) == 0:
            return c
    return 0


def _rep_spec(shape):
    nd = len(shape)
    return pl.BlockSpec(shape, lambda *_: (0,) * nd)


def _batch_spec(shape):
    nd = len(shape)
    return pl.BlockSpec((None,) + shape, lambda b: (b,) + (0,) * nd)


def _fps_body(npoint, bsz, xyzR_ref, xyzT_ref, out_ref):
    n = xyzT_ref.shape[-1]
    lane = jax.lax.broadcasted_iota(jnp.int32, (1, n), 1)
    xs = [xyzT_ref[b, 0:1, :] for b in range(bsz)]
    ys = [xyzT_ref[b, 1:2, :] for b in range(bsz)]
    zs = [xyzT_ref[b, 2:3, :] for b in range(bsz)]

    ninf = jnp.float32(-np.inf)

    def body(i, carry):
        dists, fars = carry
        new_d, new_f = [], []
        for b in range(bsz):
            out_ref[b, pl.ds(i, 1), :] = fars[b]
            sel_mask = lane == fars[b]
            cx = jnp.max(jnp.where(sel_mask, xs[b], ninf),
                         axis=1, keepdims=True)
            cy = jnp.max(jnp.where(sel_mask, ys[b], ninf),
                         axis=1, keepdims=True)
            cz = jnp.max(jnp.where(sel_mask, zs[b], ninf),
                         axis=1, keepdims=True)
            dx = xs[b] - cx
            dy = ys[b] - cy
            dz = zs[b] - cz
            dist = dx * dx + dy * dy + dz * dz
            dist_min = jnp.minimum(dists[b], dist)
            m = jnp.max(dist_min, axis=1, keepdims=True)
            sel = jnp.where(dist_min == m, lane, n)
            new_d.append(dist_min)
            new_f.append(jnp.min(sel, axis=1, keepdims=True))
        return tuple(new_d), tuple(new_f)

    init = (tuple(jnp.full((1, n), 1e10, dtype=jnp.float32)
                  for _ in range(bsz)),
            tuple(jnp.zeros((1, 1), dtype=jnp.int32) for _ in range(bsz)))
    jax.lax.fori_loop(0, npoint, body, init)


def _fps(xyz, npoint):
    b, n, _ = xyz.shape
    xyz_t = jnp.transpose(xyz, (0, 2, 1))
    out = pl.pallas_call(
        functools.partial(_fps_body, npoint, b),
        grid=(1,),
        in_specs=[_rep_spec((b, n, 3)), _rep_spec((b, 3, n))],
        out_specs=_rep_spec((b, npoint, 1)),
        out_shape=jax.ShapeDtypeStruct((b, npoint, 1), jnp.int32),
    )(xyz, xyz_t)
    return out


def _knn_body(k, bsz, has_qidx, global_ofs, *refs):
    if has_qidx:
        xyzR_ref, xyzT_ref, qidx_ref, out_ref = refs
    else:
        xyzR_ref, xyzT_ref, out_ref = refs
    n = xyzT_ref.shape[-1]
    ds = []
    for b in range(bsz):
        if has_qidx:
            nq = qidx_ref.shape[1]
            lane_q = jax.lax.broadcasted_iota(jnp.int32, (nq, n), 1)
            oh = (lane_q == qidx_ref[b]).astype(jnp.float32)
            q = jnp.dot(oh, xyzR_ref[b],
                        preferred_element_type=jnp.float32)
        else:
            nq = xyzR_ref.shape[1]
            q = xyzR_ref[b]
        dx = q[:, 0:1] - xyzT_ref[b, 0:1, :]
        dy = q[:, 1:2] - xyzT_ref[b, 1:2, :]
        dz = q[:, 2:3] - xyzT_ref[b, 2:3, :]
        ds.append(dx * dx + dy * dy + dz * dz)
    lane = jax.lax.broadcasted_iota(jnp.int32, (ds[0].shape[0], n), 1)
    big = jnp.float32(np.inf)
    for j in range(k):
        for b in range(bsz):
            m = jnp.min(ds[b], axis=1, keepdims=True)
            sel = jnp.where(ds[b] == m, lane, n)
            amin = jnp.min(sel, axis=1, keepdims=True)
            out_ref[b, :, pl.ds(j, 1)] = amin + (b * n if global_ofs else 0)
            ds[b] = jnp.where(lane == amin, big, ds[b])


def _knn_self(xyz, k, global_ofs=False):
    b, n, _ = xyz.shape
    xyz_t = jnp.transpose(xyz, (0, 2, 1))
    return pl.pallas_call(
        functools.partial(_knn_body, k, b, False, global_ofs),
        grid=(1,),
        in_specs=[_rep_spec((b, n, 3)), _rep_spec((b, 3, n))],
        out_specs=_rep_spec((b, n, k)),
        out_shape=jax.ShapeDtypeStruct((b, n, k), jnp.int32),
    )(xyz, xyz_t)


def _knn_fps(xyz, qidx, k):
    b, n, _ = xyz.shape
    nq = qidx.shape[1]
    xyz_t = jnp.transpose(xyz, (0, 2, 1))
    return pl.pallas_call(
        functools.partial(_knn_body, k, b, True, False),
        grid=(1,),
        in_specs=[_rep_spec((b, n, 3)), _rep_spec((b, 3, n)),
                  _rep_spec((b, nq, 1))],
        out_specs=_rep_spec((b, nq, k)),
        out_shape=jax.ShapeDtypeStruct((b, nq, k), jnp.int32),
    )(xyz, xyz_t, qidx)


def _tb_pre_body(xyzR_ref, f_ref, fc1w_ref, fc1b_ref, wq_ref, wk_ref,
                 wv_ref, d1w_ref, q_ref, t_ref):
    x = jnp.dot(f_ref[...], fc1w_ref[...],
                preferred_element_type=jnp.float32) + fc1b_ref[...]
    q_ref[...] = jnp.dot(x, wq_ref[...], preferred_element_type=jnp.float32)
    t_ref[:, 0:128] = jnp.dot(x, wk_ref[...],
                              preferred_element_type=jnp.float32)
    t_ref[:, 128:256] = jnp.dot(x, wv_ref[...],
                                preferred_element_type=jnp.float32)
    t_ref[:, 256:384] = jnp.dot(xyzR_ref[...], d1w_ref[...],
                                preferred_element_type=jnp.float32)


def _tb_post_body(k, tile, d1b_ref, d2w_ref, d2b_ref, g1w_ref, g1b_ref,
                  g2w_ref, g2b_ref, fc2w_ref, fc2b_ref, t_ref, q_ref,
                  knn_ref, pre_ref, out_ref):
    n = t_ref.shape[0]
    kt = k * tile
    tid = pl.program_id(1)
    pq = t_ref[pl.ds(tid * tile, tile), 256:384]
    qv = q_ref[...]
    knn = knn_ref[...]
    idx_s = jnp.concatenate([knn[:, j:j + 1] for j in range(k)], axis=0)
    lane = jax.lax.broadcasted_iota(jnp.int32, (kt, n), 1)
    oh = (lane == idx_s).astype(jnp.float32)
    g = jnp.dot(oh, t_ref[...], preferred_element_type=jnp.float32)
    pq_s = jnp.concatenate([pq] * k, axis=0)
    q_s = jnp.concatenate([qv] * k, axis=0)
    pos = jnp.maximum(pq_s - g[:, 256:384] + d1b_ref[...], 0.0)
    pos = jnp.dot(pos, d2w_ref[...],
                  preferred_element_type=jnp.float32) + d2b_ref[...]
    u = q_s - g[:, 0:128] + pos
    a = jnp.maximum(jnp.dot(u, g1w_ref[...],
                            preferred_element_type=jnp.float32)
                    + g1b_ref[...], 0.0)
    a = (jnp.dot(a, g2w_ref[...],
                 preferred_element_type=jnp.float32) + g2b_ref[...])
    a = a / _SQRT_DM
    w = g[:, 128:256] + pos
    m = a[0:tile, :]
    for j in range(1, k):
        m = jnp.maximum(m, a[j * tile:(j + 1) * tile, :])
    s = jnp.zeros((tile, _DM), jnp.float32)
    acc = jnp.zeros((tile, _DM), jnp.float32)
    for j in range(k):
        e = jnp.exp(a[j * tile:(j + 1) * tile, :] - m)
        s = s + e
        acc = acc + e * w[j * tile:(j + 1) * tile, :]
    res = acc / s
    out_ref[...] = (jnp.dot(res, fc2w_ref[...],
                            preferred_element_type=jnp.float32)
                    + fc2b_ref[...] + pre_ref[...])


def _tb_post_g_body(k, tile, d1b_ref, d2w_ref, d2b_ref, g1w_ref, g1b_ref,
                    g2w_ref, g2b_ref, fc2w_ref, fc2b_ref, g_ref, tq_ref,
                    q_ref, pre_ref, out_ref, a_sc, w_sc):
    pq = tq_ref[:, 256:384]
    qv = q_ref[...]
    for j in range(k):
        base = j * 384
        xk = g_ref[:, base:base + 128]
        xv = g_ref[:, base + 128:base + 256]
        pg = g_ref[:, base + 256:base + 384]
        pos = jnp.maximum(pq - pg + d1b_ref[...], 0.0)
        pos = jnp.dot(pos, d2w_ref[...],
                      preferred_element_type=jnp.float32) + d2b_ref[...]
        u = qv - xk + pos
        a = jnp.maximum(jnp.dot(u, g1w_ref[...],
                                preferred_element_type=jnp.float32)
                        + g1b_ref[...], 0.0)
        a = jnp.dot(a, g2w_ref[...],
                    preferred_element_type=jnp.float32) + g2b_ref[...]
        a_sc[j] = a / _SQRT_DM
        w_sc[j] = xv + pos
    m = a_sc[0]
    for j in range(1, k):
        m = jnp.maximum(m, a_sc[j])
    s = jnp.zeros((tile, _DM), jnp.float32)
    acc = jnp.zeros((tile, _DM), jnp.float32)
    for j in range(k):
        e = jnp.exp(a_sc[j] - m)
        s = s + e
        acc = acc + e * w_sc[j]
    res = acc / s
    out_ref[...] = (jnp.dot(res, fc2w_ref[...],
                            preferred_element_type=jnp.float32)
                    + fc2b_ref[...] + pre_ref[...])


def _tb(p, xyz, feats, knn, use_sc=False):
    b, n, d_in = feats.shape
    k = knn.shape[2]
    fc1w, fc1b = p['fc1']
    d1w, d1b = p['d1']
    d2w, d2b = p['d2']
    g1w, g1b = p['g1']
    g2w, g2b = p['g2']
    fc2w, fc2b = p['fc2']
    q, t = pl.pallas_call(
        _tb_pre_body,
        grid=(b,),
        in_specs=[_batch_spec((n, 3)), _batch_spec((n, d_in)),
                  _rep_spec(fc1w.shape), _rep_spec((1, _DM)),
                  _rep_spec(p['wq'].shape), _rep_spec(p['wk'].shape),
                  _rep_spec(p['wv'].shape), _rep_spec(d1w.shape)],
        out_specs=[_batch_spec((n, _DM)), _batch_spec((n, 384))],
        out_shape=[jax.ShapeDtypeStruct((b, n, _DM), jnp.float32),
                   jax.ShapeDtypeStruct((b, n, 384), jnp.float32)],
    )(xyz, feats, fc1w, fc1b.reshape(1, -1), p['wq'], p['wk'], p['wv'], d1w)

    if use_sc:
        chunk = _sc_chunk(b * n * k)
        g = _sc_gather(t.reshape(b * n, 384), knn.reshape(-1), chunk)
        gr = g.reshape(b, n, k * 384)
        tile = min(n, 128)
        nt = n // tile
        out = pl.pallas_call(
            functools.partial(_tb_post_g_body, k, tile),
            grid=(b, nt),
            in_specs=[_rep_spec((1, _DM)), _rep_spec(d2w.shape),
                      _rep_spec((1, _DM)), _rep_spec(g1w.shape),
                      _rep_spec((1, _DM)), _rep_spec(g2w.shape),
                      _rep_spec((1, _DM)), _rep_spec(fc2w.shape),
                      _rep_spec((1, d_in)),
                      pl.BlockSpec((None, tile, k * 384),
                                   lambda b_, t_: (b_, t_, 0)),
                      pl.BlockSpec((None, tile, 384),
                                   lambda b_, t_: (b_, t_, 0)),
                      pl.BlockSpec((None, tile, _DM),
                                   lambda b_, t_: (b_, t_, 0)),
                      pl.BlockSpec((None, tile, d_in),
                                   lambda b_, t_: (b_, t_, 0))],
            out_specs=pl.BlockSpec((None, tile, d_in),
                                   lambda b_, t_: (b_, t_, 0)),
            out_shape=jax.ShapeDtypeStruct((b, n, d_in), jnp.float32),
            scratch_shapes=[pltpu.VMEM((k, tile, _DM), jnp.float32),
                            pltpu.VMEM((k, tile, _DM), jnp.float32)],
        )(d1b.reshape(1, -1), d2w, d2b.reshape(1, -1), g1w,
          g1b.reshape(1, -1), g2w, g2b.reshape(1, -1), fc2w,
          fc2b.reshape(1, -1), gr, t, q, feats)
        return out

    tile = min(n, max(1024 // k, 8))
    nt = n // tile
    out = pl.pallas_call(
        functools.partial(_tb_post_body, k, tile),
        grid=(b, nt),
        in_specs=[_rep_spec((1, _DM)), _rep_spec(d2w.shape),
                  _rep_spec((1, _DM)), _rep_spec(g1w.shape),
                  _rep_spec((1, _DM)), _rep_spec(g2w.shape),
                  _rep_spec((1, _DM)), _rep_spec(fc2w.shape),
                  _rep_spec((1, d_in)),
                  pl.BlockSpec((None, n, 384), lambda b_, t_: (b_, 0, 0)),
                  pl.BlockSpec((None, tile, _DM), lambda b_, t_: (b_, t_, 0)),
                  pl.BlockSpec((None, tile, k), lambda b_, t_: (b_, t_, 0)),
                  pl.BlockSpec((None, tile, d_in), lambda b_, t_: (b_, t_, 0))],
        out_specs=pl.BlockSpec((None, tile, d_in), lambda b_, t_: (b_, t_, 0)),
        out_shape=jax.ShapeDtypeStruct((b, n, d_in), jnp.float32),
    )(d1b.reshape(1, -1), d2w, d2b.reshape(1, -1), g1w, g1b.reshape(1, -1),
      g2w, g2b.reshape(1, -1), fc2w, fc2b.reshape(1, -1), t, q, knn, feats)
    return out


def _td_body(k, xyzR_ref, f_ref, fps_ref, knn_ref, l1wx_ref, l1wf_ref,
             l1b_ref, l2w_ref, l2b_ref, nxyz_ref, out_ref):
    n = xyzR_ref.shape[0]
    npt = fps_ref.shape[0]
    c_out = l2w_ref.shape[0]
    lane = jax.lax.broadcasted_iota(jnp.int32, (npt, n), 1)
    oh_fps = (lane == fps_ref[:, :]).astype(jnp.float32)
    new_xyz = jnp.dot(oh_fps, xyzR_ref[...],
                      preferred_element_type=jnp.float32)
    nxyz_ref[...] = new_xyz
    knn = knn_ref[...]
    m = jnp.full((npt, c_out), -jnp.inf, jnp.float32)
    for j in range(k):
        idx = knn[:, j:j + 1]
        oh = (lane == idx).astype(jnp.float32)
        gx = jnp.dot(oh, xyzR_ref[...],
                     preferred_element_type=jnp.float32) - new_xyz
        gf = jnp.dot(oh, f_ref[...], preferred_element_type=jnp.float32)
        h = (jnp.dot(gx, l1wx_ref[...], preferred_element_type=jnp.float32)
             + jnp.dot(gf, l1wf_ref[...], preferred_element_type=jnp.float32)
             + l1b_ref[...])
        h = jnp.maximum(h, 0.0)
        h = jnp.dot(h, l2w_ref[...],
                    preferred_element_type=jnp.float32) + l2b_ref[...]
        h = jnp.maximum(h, 0.0)
        m = jnp.maximum(m, h)
    out_ref[...] = m


def _td(p, xyz, feats, fps, knn):
    b, n, c_in = feats.shape
    npt = fps.shape[1]
    k = knn.shape[2]
    l1w, l1b = p['l1']
    l2w, l2b = p['l2']
    c_out = l2w.shape[1]
    nxyz, f_out = pl.pallas_call(
        functools.partial(_td_body, k),
        grid=(b,),
        in_specs=[_batch_spec((n, 3)), _batch_spec((n, c_in)),
                  _batch_spec((npt, 1)), _batch_spec((npt, k)),
                  _rep_spec((3, c_out)), _rep_spec((c_in, c_out)),
                  _rep_spec((1, c_out)), _rep_spec(l2w.shape),
                  _rep_spec((1, c_out))],
        out_specs=[_batch_spec((npt, 3)), _batch_spec((npt, c_out))],
        out_shape=[jax.ShapeDtypeStruct((b, npt, 3), jnp.float32),
                   jax.ShapeDtypeStruct((b, npt, c_out), jnp.float32)],
    )(xyz, feats, fps, knn, l1w[:3], l1w[3:], l1b.reshape(1, -1),
      l2w, l2b.reshape(1, -1))
    return nxyz, f_out


def _tu_body(fc_ref, xycR_ref, xycT_ref, ff_ref, xyf_ref, w1_ref, b1_ref,
             w2_ref, b2_ref, out_ref):
    nc = xycR_ref.shape[0]
    nf = xyf_ref.shape[0]
    f1 = jnp.maximum(jnp.dot(fc_ref[...], w1_ref[...],
                             preferred_element_type=jnp.float32)
                     + b1_ref[...], 0.0)
    f2 = jnp.maximum(jnp.dot(ff_ref[...], w2_ref[...],
                             preferred_element_type=jnp.float32)
                     + b2_ref[...], 0.0)
    dx = xyf_ref[:, 0:1] - xycT_ref[0:1, :]
    dy = xyf_ref[:, 1:2] - xycT_ref[1:2, :]
    dz = xyf_ref[:, 2:3] - xycT_ref[2:3, :]
    d = dx * dx + dy * dy + dz * dz
    lane = jax.lax.broadcasted_iota(jnp.int32, (nf, nc), 1)
    big = jnp.float32(np.inf)
    ws = []
    idxs = []
    for j in range(3):
        m = jnp.min(d, axis=1, keepdims=True)
        sel = jnp.where(d == m, lane, nc)
        amin = jnp.min(sel, axis=1, keepdims=True)
        ws.append(1.0 / jnp.maximum(m, 1e-10))
        idxs.append(amin)
        d = jnp.where(lane == amin, big, d)
    wsum = (ws[0] + ws[1]) + ws[2]
    acc = None
    for j in range(3):
        oh = (lane == idxs[j]).astype(jnp.float32)
        fj = jnp.dot(oh, f1, preferred_element_type=jnp.float32)
        term = (ws[j] / wsum) * fj
        acc = term if acc is None else acc + term
    out_ref[...] = acc + f2


def _tu(p, f_coarse, xyz_coarse, f_fine, xyz_fine):
    b, nc, _ = xyz_coarse.shape
    nf = xyz_fine.shape[1]
    w1, b1 = p['fc1']
    w2, b2 = p['fc2']
    d = w1.shape[1]
    xyc_t = jnp.transpose(xyz_coarse, (0, 2, 1))
    return pl.pallas_call(
        _tu_body,
        grid=(b,),
        in_specs=[_batch_spec(f_coarse.shape[1:]), _batch_spec((nc, 3)),
                  _batch_spec((3, nc)), _batch_spec(f_fine.shape[1:]),
                  _batch_spec((nf, 3)), _rep_spec(w1.shape),
                  _rep_spec((1, d)), _rep_spec(w2.shape), _rep_spec((1, d))],
        out_specs=_batch_spec((nf, d)),
        out_shape=jax.ShapeDtypeStruct((b, nf, d), jnp.float32),
    )(f_coarse, xyz_coarse, xyc_t, f_fine, xyz_fine, w1,
      b1.reshape(1, -1), w2, b2.reshape(1, -1))


def _mlp_body(relus, nlayer, *refs):
    x_ref = refs[0]
    out_ref = refs[-1]
    h = x_ref[...]
    for i in range(nlayer):
        w_ref = refs[1 + 2 * i]
        b_ref = refs[2 + 2 * i]
        h = jnp.dot(h, w_ref[...],
                    preferred_element_type=jnp.float32) + b_ref[...]
        if relus[i]:
            h = jnp.maximum(h, 0.0)
    out_ref[...] = h


def _mlp(x, layers, relus):
    b, n, _ = x.shape
    nlayer = len(layers)
    args = [x]
    specs = [_batch_spec(x.shape[1:])]
    for (w, bias) in layers:
        args.append(w)
        args.append(bias.reshape(1, -1))
        specs.append(_rep_spec(w.shape))
        specs.append(_rep_spec((1, w.shape[1])))
    d_out = layers[-1][0].shape[1]
    return pl.pallas_call(
        functools.partial(_mlp_body, relus, nlayer),
        grid=(b,),
        in_specs=specs,
        out_specs=_batch_spec((n, d_out)),
        out_shape=jax.ShapeDtypeStruct((b, n, d_out), jnp.float32),
    )(*args)


def _tb_stage(p, xyz, feats):
    n = xyz.shape[1]
    k = min(_KP, n)
    use_sc = False
    knn = _knn_self(xyz, k, global_ofs=use_sc)
    return _tb(p, xyz, feats, knn, use_sc=use_sc)


def kernel(x, params):
    xb = jnp.transpose(x, (0, 2, 1))
    xyz = xb
    f = _mlp(xb, [params['bb_fc1a'], params['bb_fc1b']], [True, False])
    f = _tb_stage(params['bb_tb0'], xyz, f)
    fac = [(f, xyz)]
    npts = xyz.shape[1]
    for i in range(4):
        npts //= 4
        fps = _fps(xyz, npts)
        knn_d = _knn_fps(xyz, fps, _KP)
        xyz, f = _td(params['bb_td'][i], xyz, f, fps, knn_d)
        f = _tb_stage(params['bb_tbs'][i], xyz, f)
        fac.append((f, xyz))
    feature, coord = fac[-1]
    h = _mlp(feature, [params['mlp2a'], params['mlp2b'], params['mlp2c']],
             [True, True, False])
    feature = _tb_stage(params['t2'], coord, h)
    for i in range(4):
        f_fine, c_fine = fac[-i - 2]
        feature = _tu(params['tu'][i], feature, coord, f_fine, c_fine)
        coord = c_fine
        feature = _tb_stage(params['tbu'][i], coord, feature)
    h = _mlp(feature, [params['mlp3a'], params['mlp3b'], params['mlp3c']],
             [True, True, False])
    return h


# speedup vs baseline: 1.8559x; 1.0022x over previous
import functools

import jax
import jax.numpy as jnp
import numpy as np
from jax.experimental import pallas as pl
from jax.experimental.pallas import tpu as pltpu
from jax.experimental.pallas import tpu_sc as plsc

_B = 2
_KP = 16
_DM = 128
_SQRT_DM = np.float32(np.sqrt(128.0))

_SC_NC = 2
_SC_NS = 16
_SC_NW = _SC_NC * _SC_NS


def _sc_gather(table, idx, chunk):
    bn = idx.shape[0]
    d = table.shape[1]
    per_w = bn // (chunk * _SC_NW)
    mesh = plsc.VectorSubcoreMesh(core_axis_name="c", subcore_axis_name="s",
                                  num_cores=_SC_NC, num_subcores=_SC_NS)

    def body(table_hbm, idx_hbm, out_hbm, idx_v, rows_v, sem):
        wid = jax.lax.axis_index("s") * _SC_NC + jax.lax.axis_index("c")
        for j in range(per_w):
            base = (wid * per_w + j) * chunk
            pltpu.sync_copy(idx_hbm.at[pl.ds(base, chunk)], idx_v)
            pltpu.async_copy(table_hbm.at[idx_v], rows_v, sem).wait()
            pltpu.sync_copy(rows_v, out_hbm.at[pl.ds(base, chunk)])

    f = pl.kernel(
        body,
        out_type=jax.ShapeDtypeStruct((bn, d), jnp.float32),
        mesh=mesh,
        scratch_types=[pltpu.VMEM((chunk,), jnp.int32),
                       pltpu.VMEM((chunk, d), jnp.float32),
                       pltpu.SemaphoreType.DMA],
    )
    return f(table, idx)


def _sc_chunk(bn):
    for c in (128, 64, 32, 16, 8):
        if bn % (c * _SC_NW) == 0:
            return c
    return 0


def _rep_spec(shape):
    nd = len(shape)
    return pl.BlockSpec(shape, lambda *_: (0,) * nd)


def _batch_spec(shape):
    nd = len(shape)
    return pl.BlockSpec((None,) + shape, lambda b: (b,) + (0,) * nd)


def _fps_body(npoint, bsz, xyzR_ref, xyzT_ref, out_ref):
    n = xyzT_ref.shape[-1]
    lane = jax.lax.broadcasted_iota(jnp.int32, (1, n), 1)
    xs = [xyzT_ref[b, 0:1, :] for b in range(bsz)]
    ys = [xyzT_ref[b, 1:2, :] for b in range(bsz)]
    zs = [xyzT_ref[b, 2:3, :] for b in range(bsz)]

    ninf = jnp.float32(-np.inf)

    def body(i, carry):
        dists, fars = carry
        new_d, new_f = [], []
        for b in range(bsz):
            out_ref[b, pl.ds(i, 1), :] = fars[b]
            sel_mask = lane == fars[b]
            cx = jnp.max(jnp.where(sel_mask, xs[b], ninf),
                         axis=1, keepdims=True)
            cy = jnp.max(jnp.where(sel_mask, ys[b], ninf),
                         axis=1, keepdims=True)
            cz = jnp.max(jnp.where(sel_mask, zs[b], ninf),
                         axis=1, keepdims=True)
            dx = xs[b] - cx
            dy = ys[b] - cy
            dz = zs[b] - cz
            dist = dx * dx + dy * dy + dz * dz
            dist_min = jnp.minimum(dists[b], dist)
            m = jnp.max(dist_min, axis=1, keepdims=True)
            sel = jnp.where(dist_min == m, lane, n)
            new_d.append(dist_min)
            new_f.append(jnp.min(sel, axis=1, keepdims=True))
        return tuple(new_d), tuple(new_f)

    init = (tuple(jnp.full((1, n), 1e10, dtype=jnp.float32)
                  for _ in range(bsz)),
            tuple(jnp.zeros((1, 1), dtype=jnp.int32) for _ in range(bsz)))
    jax.lax.fori_loop(0, npoint, body, init)


def _fps(xyz, npoint):
    b, n, _ = xyz.shape
    xyz_t = jnp.transpose(xyz, (0, 2, 1))
    out = pl.pallas_call(
        functools.partial(_fps_body, npoint, b),
        grid=(1,),
        in_specs=[_rep_spec((b, n, 3)), _rep_spec((b, 3, n))],
        out_specs=_rep_spec((b, npoint, 1)),
        out_shape=jax.ShapeDtypeStruct((b, npoint, 1), jnp.int32),
    )(xyz, xyz_t)
    return out


def _knn_body(k, bsz, has_qidx, global_ofs, *refs):
    if has_qidx:
        xyzR_ref, xyzT_ref, qidx_ref, out_ref = refs
    else:
        xyzR_ref, xyzT_ref, out_ref = refs
    n = xyzT_ref.shape[-1]
    ds = []
    for b in range(bsz):
        if has_qidx:
            nq = qidx_ref.shape[1]
            lane_q = jax.lax.broadcasted_iota(jnp.int32, (nq, n), 1)
            oh = (lane_q == qidx_ref[b]).astype(jnp.float32)
            q = jnp.dot(oh, xyzR_ref[b],
                        preferred_element_type=jnp.float32)
        else:
            nq = xyzR_ref.shape[1]
            q = xyzR_ref[b]
        dx = q[:, 0:1] - xyzT_ref[b, 0:1, :]
        dy = q[:, 1:2] - xyzT_ref[b, 1:2, :]
        dz = q[:, 2:3] - xyzT_ref[b, 2:3, :]
        ds.append(dx * dx + dy * dy + dz * dz)
    lane = jax.lax.broadcasted_iota(jnp.int32, (ds[0].shape[0], n), 1)
    big = jnp.float32(np.inf)
    for j in range(k):
        for b in range(bsz):
            m = jnp.min(ds[b], axis=1, keepdims=True)
            sel = jnp.where(ds[b] == m, lane, n)
            amin = jnp.min(sel, axis=1, keepdims=True)
            out_ref[b, :, pl.ds(j, 1)] = amin + (b * n if global_ofs else 0)
            ds[b] = jnp.where(lane == amin, big, ds[b])


def _knn_self(xyz, k, global_ofs=False):
    b, n, _ = xyz.shape
    xyz_t = jnp.transpose(xyz, (0, 2, 1))
    return pl.pallas_call(
        functools.partial(_knn_body, k, b, False, global_ofs),
        grid=(1,),
        in_specs=[_rep_spec((b, n, 3)), _rep_spec((b, 3, n))],
        out_specs=_rep_spec((b, n, k)),
        out_shape=jax.ShapeDtypeStruct((b, n, k), jnp.int32),
    )(xyz, xyz_t)


def _knn_fps(xyz, qidx, k):
    b, n, _ = xyz.shape
    nq = qidx.shape[1]
    xyz_t = jnp.transpose(xyz, (0, 2, 1))
    return pl.pallas_call(
        functools.partial(_knn_body, k, b, True, False),
        grid=(1,),
        in_specs=[_rep_spec((b, n, 3)), _rep_spec((b, 3, n)),
                  _rep_spec((b, nq, 1))],
        out_specs=_rep_spec((b, nq, k)),
        out_shape=jax.ShapeDtypeStruct((b, nq, k), jnp.int32),
    )(xyz, xyz_t, qidx)


def _tb_pre_body(bsz, xyzR_ref, f_ref, fc1w_ref, fc1b_ref, wq_ref, wk_ref,
                 wv_ref, d1w_ref, q_ref, t_ref):
    for b in range(bsz):
        x = jnp.dot(f_ref[b], fc1w_ref[...],
                    preferred_element_type=jnp.float32) + fc1b_ref[...]
        q_ref[b] = jnp.dot(x, wq_ref[...],
                           preferred_element_type=jnp.float32)
        t_ref[b, :, 0:128] = jnp.dot(x, wk_ref[...],
                                     preferred_element_type=jnp.float32)
        t_ref[b, :, 128:256] = jnp.dot(x, wv_ref[...],
                                       preferred_element_type=jnp.float32)
        t_ref[b, :, 256:384] = jnp.dot(xyzR_ref[b], d1w_ref[...],
                                       preferred_element_type=jnp.float32)


def _tb_post_body(k, tile, d1b_ref, d2w_ref, d2b_ref, g1w_ref, g1b_ref,
                  g2w_ref, g2b_ref, fc2w_ref, fc2b_ref, t_ref, q_ref,
                  knn_ref, pre_ref, out_ref):
    n = t_ref.shape[0]
    kt = k * tile
    tid = pl.program_id(1)
    pq = t_ref[pl.ds(tid * tile, tile), 256:384]
    qv = q_ref[...]
    knn = knn_ref[...]
    idx_s = jnp.concatenate([knn[:, j:j + 1] for j in range(k)], axis=0)
    lane = jax.lax.broadcasted_iota(jnp.int32, (kt, n), 1)
    oh = (lane == idx_s).astype(jnp.float32)
    g = jnp.dot(oh, t_ref[...], preferred_element_type=jnp.float32)
    pq_s = jnp.concatenate([pq] * k, axis=0)
    q_s = jnp.concatenate([qv] * k, axis=0)
    pos = jnp.maximum(pq_s - g[:, 256:384] + d1b_ref[...], 0.0)
    pos = jnp.dot(pos, d2w_ref[...],
                  preferred_element_type=jnp.float32) + d2b_ref[...]
    u = q_s - g[:, 0:128] + pos
    a = jnp.maximum(jnp.dot(u, g1w_ref[...],
                            preferred_element_type=jnp.float32)
                    + g1b_ref[...], 0.0)
    a = (jnp.dot(a, g2w_ref[...],
                 preferred_element_type=jnp.float32) + g2b_ref[...])
    a = a / _SQRT_DM
    w = g[:, 128:256] + pos
    m = a[0:tile, :]
    for j in range(1, k):
        m = jnp.maximum(m, a[j * tile:(j + 1) * tile, :])
    s = jnp.zeros((tile, _DM), jnp.float32)
    acc = jnp.zeros((tile, _DM), jnp.float32)
    for j in range(k):
        e = jnp.exp(a[j * tile:(j + 1) * tile, :] - m)
        s = s + e
        acc = acc + e * w[j * tile:(j + 1) * tile, :]
    res = acc / s
    out_ref[...] = (jnp.dot(res, fc2w_ref[...],
                            preferred_element_type=jnp.float32)
                    + fc2b_ref[...] + pre_ref[...])


def _tb_post_g_body(k, tile, d1b_ref, d2w_ref, d2b_ref, g1w_ref, g1b_ref,
                    g2w_ref, g2b_ref, fc2w_ref, fc2b_ref, g_ref, tq_ref,
                    q_ref, pre_ref, out_ref, a_sc, w_sc):
    pq = tq_ref[:, 256:384]
    qv = q_ref[...]
    for j in range(k):
        base = j * 384
        xk = g_ref[:, base:base + 128]
        xv = g_ref[:, base + 128:base + 256]
        pg = g_ref[:, base + 256:base + 384]
        pos = jnp.maximum(pq - pg + d1b_ref[...], 0.0)
        pos = jnp.dot(pos, d2w_ref[...],
                      preferred_element_type=jnp.float32) + d2b_ref[...]
        u = qv - xk + pos
        a = jnp.maximum(jnp.dot(u, g1w_ref[...],
                                preferred_element_type=jnp.float32)
                        + g1b_ref[...], 0.0)
        a = jnp.dot(a, g2w_ref[...],
                    preferred_element_type=jnp.float32) + g2b_ref[...]
        a_sc[j] = a / _SQRT_DM
        w_sc[j] = xv + pos
    m = a_sc[0]
    for j in range(1, k):
        m = jnp.maximum(m, a_sc[j])
    s = jnp.zeros((tile, _DM), jnp.float32)
    acc = jnp.zeros((tile, _DM), jnp.float32)
    for j in range(k):
        e = jnp.exp(a_sc[j] - m)
        s = s + e
        acc = acc + e * w_sc[j]
    res = acc / s
    out_ref[...] = (jnp.dot(res, fc2w_ref[...],
                            preferred_element_type=jnp.float32)
                    + fc2b_ref[...] + pre_ref[...])


def _tb(p, xyz, feats, knn, use_sc=False):
    b, n, d_in = feats.shape
    k = knn.shape[2]
    fc1w, fc1b = p['fc1']
    d1w, d1b = p['d1']
    d2w, d2b = p['d2']
    g1w, g1b = p['g1']
    g2w, g2b = p['g2']
    fc2w, fc2b = p['fc2']
    q, t = pl.pallas_call(
        functools.partial(_tb_pre_body, b),
        grid=(1,),
        in_specs=[_rep_spec((b, n, 3)), _rep_spec((b, n, d_in)),
                  _rep_spec(fc1w.shape), _rep_spec((1, _DM)),
                  _rep_spec(p['wq'].shape), _rep_spec(p['wk'].shape),
                  _rep_spec(p['wv'].shape), _rep_spec(d1w.shape)],
        out_specs=[_rep_spec((b, n, _DM)), _rep_spec((b, n, 384))],
        out_shape=[jax.ShapeDtypeStruct((b, n, _DM), jnp.float32),
                   jax.ShapeDtypeStruct((b, n, 384), jnp.float32)],
    )(xyz, feats, fc1w, fc1b.reshape(1, -1), p['wq'], p['wk'], p['wv'], d1w)

    if use_sc:
        chunk = _sc_chunk(b * n * k)
        g = _sc_gather(t.reshape(b * n, 384), knn.reshape(-1), chunk)
        gr = g.reshape(b, n, k * 384)
        tile = min(n, 128)
        nt = n // tile
        out = pl.pallas_call(
            functools.partial(_tb_post_g_body, k, tile),
            grid=(b, nt),
            in_specs=[_rep_spec((1, _DM)), _rep_spec(d2w.shape),
                      _rep_spec((1, _DM)), _rep_spec(g1w.shape),
                      _rep_spec((1, _DM)), _rep_spec(g2w.shape),
                      _rep_spec((1, _DM)), _rep_spec(fc2w.shape),
                      _rep_spec((1, d_in)),
                      pl.BlockSpec((None, tile, k * 384),
                                   lambda b_, t_: (b_, t_, 0)),
                      pl.BlockSpec((None, tile, 384),
                                   lambda b_, t_: (b_, t_, 0)),
                      pl.BlockSpec((None, tile, _DM),
                                   lambda b_, t_: (b_, t_, 0)),
                      pl.BlockSpec((None, tile, d_in),
                                   lambda b_, t_: (b_, t_, 0))],
            out_specs=pl.BlockSpec((None, tile, d_in),
                                   lambda b_, t_: (b_, t_, 0)),
            out_shape=jax.ShapeDtypeStruct((b, n, d_in), jnp.float32),
            scratch_shapes=[pltpu.VMEM((k, tile, _DM), jnp.float32),
                            pltpu.VMEM((k, tile, _DM), jnp.float32)],
        )(d1b.reshape(1, -1), d2w, d2b.reshape(1, -1), g1w,
          g1b.reshape(1, -1), g2w, g2b.reshape(1, -1), fc2w,
          fc2b.reshape(1, -1), gr, t, q, feats)
        return out

    tile = min(n, max(1024 // k, 8))
    nt = n // tile
    out = pl.pallas_call(
        functools.partial(_tb_post_body, k, tile),
        grid=(b, nt),
        in_specs=[_rep_spec((1, _DM)), _rep_spec(d2w.shape),
                  _rep_spec((1, _DM)), _rep_spec(g1w.shape),
                  _rep_spec((1, _DM)), _rep_spec(g2w.shape),
                  _rep_spec((1, _DM)), _rep_spec(fc2w.shape),
                  _rep_spec((1, d_in)),
                  pl.BlockSpec((None, n, 384), lambda b_, t_: (b_, 0, 0)),
                  pl.BlockSpec((None, tile, _DM), lambda b_, t_: (b_, t_, 0)),
                  pl.BlockSpec((None, tile, k), lambda b_, t_: (b_, t_, 0)),
                  pl.BlockSpec((None, tile, d_in), lambda b_, t_: (b_, t_, 0))],
        out_specs=pl.BlockSpec((None, tile, d_in), lambda b_, t_: (b_, t_, 0)),
        out_shape=jax.ShapeDtypeStruct((b, n, d_in), jnp.float32),
    )(d1b.reshape(1, -1), d2w, d2b.reshape(1, -1), g1w, g1b.reshape(1, -1),
      g2w, g2b.reshape(1, -1), fc2w, fc2b.reshape(1, -1), t, q, knn, feats)
    return out


def _td_body(k, xyzR_ref, f_ref, fps_ref, knn_ref, l1wx_ref, l1wf_ref,
             l1b_ref, l2w_ref, l2b_ref, nxyz_ref, out_ref):
    n = xyzR_ref.shape[0]
    npt = fps_ref.shape[0]
    c_out = l2w_ref.shape[0]
    lane = jax.lax.broadcasted_iota(jnp.int32, (npt, n), 1)
    oh_fps = (lane == fps_ref[:, :]).astype(jnp.float32)
    new_xyz = jnp.dot(oh_fps, xyzR_ref[...],
                      preferred_element_type=jnp.float32)
    nxyz_ref[...] = new_xyz
    knn = knn_ref[...]
    m = jnp.full((npt, c_out), -jnp.inf, jnp.float32)
    for j in range(k):
        idx = knn[:, j:j + 1]
        oh = (lane == idx).astype(jnp.float32)
        gx = jnp.dot(oh, xyzR_ref[...],
                     preferred_element_type=jnp.float32) - new_xyz
        gf = jnp.dot(oh, f_ref[...], preferred_element_type=jnp.float32)
        h = (jnp.dot(gx, l1wx_ref[...], preferred_element_type=jnp.float32)
             + jnp.dot(gf, l1wf_ref[...], preferred_element_type=jnp.float32)
             + l1b_ref[...])
        h = jnp.maximum(h, 0.0)
        h = jnp.dot(h, l2w_ref[...],
                    preferred_element_type=jnp.float32) + l2b_ref[...]
        h = jnp.maximum(h, 0.0)
        m = jnp.maximum(m, h)
    out_ref[...] = m


def _td(p, xyz, feats, fps, knn):
    b, n, c_in = feats.shape
    npt = fps.shape[1]
    k = knn.shape[2]
    l1w, l1b = p['l1']
    l2w, l2b = p['l2']
    c_out = l2w.shape[1]
    nxyz, f_out = pl.pallas_call(
        functools.partial(_td_body, k),
        grid=(b,),
        in_specs=[_batch_spec((n, 3)), _batch_spec((n, c_in)),
                  _batch_spec((npt, 1)), _batch_spec((npt, k)),
                  _rep_spec((3, c_out)), _rep_spec((c_in, c_out)),
                  _rep_spec((1, c_out)), _rep_spec(l2w.shape),
                  _rep_spec((1, c_out))],
        out_specs=[_batch_spec((npt, 3)), _batch_spec((npt, c_out))],
        out_shape=[jax.ShapeDtypeStruct((b, npt, 3), jnp.float32),
                   jax.ShapeDtypeStruct((b, npt, c_out), jnp.float32)],
    )(xyz, feats, fps, knn, l1w[:3], l1w[3:], l1b.reshape(1, -1),
      l2w, l2b.reshape(1, -1))
    return nxyz, f_out


def _tu_body(fc_ref, xycR_ref, xycT_ref, ff_ref, xyf_ref, w1_ref, b1_ref,
             w2_ref, b2_ref, out_ref):
    nc = xycR_ref.shape[0]
    nf = xyf_ref.shape[0]
    f1 = jnp.maximum(jnp.dot(fc_ref[...], w1_ref[...],
                             preferred_element_type=jnp.float32)
                     + b1_ref[...], 0.0)
    f2 = jnp.maximum(jnp.dot(ff_ref[...], w2_ref[...],
                             preferred_element_type=jnp.float32)
                     + b2_ref[...], 0.0)
    dx = xyf_ref[:, 0:1] - xycT_ref[0:1, :]
    dy = xyf_ref[:, 1:2] - xycT_ref[1:2, :]
    dz = xyf_ref[:, 2:3] - xycT_ref[2:3, :]
    d = dx * dx + dy * dy + dz * dz
    lane = jax.lax.broadcasted_iota(jnp.int32, (nf, nc), 1)
    big = jnp.float32(np.inf)
    ws = []
    idxs = []
    for j in range(3):
        m = jnp.min(d, axis=1, keepdims=True)
        sel = jnp.where(d == m, lane, nc)
        amin = jnp.min(sel, axis=1, keepdims=True)
        ws.append(1.0 / jnp.maximum(m, 1e-10))
        idxs.append(amin)
        d = jnp.where(lane == amin, big, d)
    wsum = (ws[0] + ws[1]) + ws[2]
    acc = None
    for j in range(3):
        oh = (lane == idxs[j]).astype(jnp.float32)
        fj = jnp.dot(oh, f1, preferred_element_type=jnp.float32)
        term = (ws[j] / wsum) * fj
        acc = term if acc is None else acc + term
    out_ref[...] = acc + f2


def _tu(p, f_coarse, xyz_coarse, f_fine, xyz_fine):
    b, nc, _ = xyz_coarse.shape
    nf = xyz_fine.shape[1]
    w1, b1 = p['fc1']
    w2, b2 = p['fc2']
    d = w1.shape[1]
    xyc_t = jnp.transpose(xyz_coarse, (0, 2, 1))
    return pl.pallas_call(
        _tu_body,
        grid=(b,),
        in_specs=[_batch_spec(f_coarse.shape[1:]), _batch_spec((nc, 3)),
                  _batch_spec((3, nc)), _batch_spec(f_fine.shape[1:]),
                  _batch_spec((nf, 3)), _rep_spec(w1.shape),
                  _rep_spec((1, d)), _rep_spec(w2.shape), _rep_spec((1, d))],
        out_specs=_batch_spec((nf, d)),
        out_shape=jax.ShapeDtypeStruct((b, nf, d), jnp.float32),
    )(f_coarse, xyz_coarse, xyc_t, f_fine, xyz_fine, w1,
      b1.reshape(1, -1), w2, b2.reshape(1, -1))


def _mlp_body(relus, nlayer, bsz, *refs):
    x_ref = refs[0]
    out_ref = refs[-1]
    for b in range(bsz):
        h = x_ref[b]
        for i in range(nlayer):
            w_ref = refs[1 + 2 * i]
            b_ref = refs[2 + 2 * i]
            h = jnp.dot(h, w_ref[...],
                        preferred_element_type=jnp.float32) + b_ref[...]
            if relus[i]:
                h = jnp.maximum(h, 0.0)
        out_ref[b] = h


def _mlp(x, layers, relus):
    b, n, _ = x.shape
    nlayer = len(layers)
    args = [x]
    specs = [_rep_spec(x.shape)]
    for (w, bias) in layers:
        args.append(w)
        args.append(bias.reshape(1, -1))
        specs.append(_rep_spec(w.shape))
        specs.append(_rep_spec((1, w.shape[1])))
    d_out = layers[-1][0].shape[1]
    return pl.pallas_call(
        functools.partial(_mlp_body, relus, nlayer, b),
        grid=(1,),
        in_specs=specs,
        out_specs=_rep_spec((b, n, d_out)),
        out_shape=jax.ShapeDtypeStruct((b, n, d_out), jnp.float32),
    )(*args)


def _tb_stage(p, xyz, feats):
    n = xyz.shape[1]
    k = min(_KP, n)
    use_sc = False
    knn = _knn_self(xyz, k, global_ofs=use_sc)
    return _tb(p, xyz, feats, knn, use_sc=use_sc)


def kernel(x, params):
    xb = jnp.transpose(x, (0, 2, 1))
    xyz = xb
    f = _mlp(xb, [params['bb_fc1a'], params['bb_fc1b']], [True, False])
    f = _tb_stage(params['bb_tb0'], xyz, f)
    fac = [(f, xyz)]
    npts = xyz.shape[1]
    for i in range(4):
        npts //= 4
        fps = _fps(xyz, npts)
        knn_d = _knn_fps(xyz, fps, _KP)
        xyz, f = _td(params['bb_td'][i], xyz, f, fps, knn_d)
        f = _tb_stage(params['bb_tbs'][i], xyz, f)
        fac.append((f, xyz))
    feature, coord = fac[-1]
    h = _mlp(feature, [params['mlp2a'], params['mlp2b'], params['mlp2c']],
             [True, True, False])
    feature = _tb_stage(params['t2'], coord, h)
    for i in range(4):
        f_fine, c_fine = fac[-i - 2]
        feature = _tu(params['tu'][i], feature, coord, f_fine, c_fine)
        coord = c_fine
        feature = _tb_stage(params['tbu'][i], coord, feature)
    h = _mlp(feature, [params['mlp3a'], params['mlp3b'], params['mlp3c']],
             [True, True, False])
    return h
